# baseline (speedup 1.0000x reference)
import jax
import jax.numpy as jnp
from jax import lax
from jax.experimental import pallas as pl
from jax.experimental.pallas import tpu as pltpu

_NCLS = 20
_STRIDES = (8, 16, 32)
_VMEM_LIMIT = 52 * 1024 * 1024


def _make_tiled_conv_call(K, Cin, Cout, relu, H, W, bh, name):
    T = H // bh
    Wp = W + K - 1

    def body(a_ref, b_ref, w_ref, bias_ref, out_ref):
        xin = jnp.concatenate([a_ref[0], b_ref[0][:K - 1]], axis=0)
        acc = jnp.zeros((bh * W, Cout), jnp.float32)
        for kh in range(K):
            for kw in range(K):
                xs = xin[kh:kh + bh, kw:kw + W, :].reshape(bh * W, Cin)
                acc = acc + jnp.dot(xs, w_ref[kh * K + kw],
                                    preferred_element_type=jnp.float32)
        y = acc + bias_ref[...]
        if relu:
            y = jnp.maximum(y, 0.0)
        out_ref[0] = y

    def call(x, w, bias):
        B = x.shape[0]
        return pl.pallas_call(
            body,
            out_shape=jax.ShapeDtypeStruct((B, H * W, Cout), jnp.float32),
            grid=(B, T),
            in_specs=[
                pl.BlockSpec((1, bh, Wp, Cin), lambda b, t: (b, t, 0, 0)),
                pl.BlockSpec((1, bh, Wp, Cin), lambda b, t: (b, t + 1, 0, 0)),
                pl.BlockSpec((K * K, Cin, Cout), lambda b, t: (0, 0, 0)),
                pl.BlockSpec((1, Cout), lambda b, t: (0, 0)),
            ],
            out_specs=pl.BlockSpec((1, bh * W, Cout), lambda b, t: (b, t, 0)),
            compiler_params=pltpu.CompilerParams(
                dimension_semantics=("parallel", "arbitrary"),
                vmem_limit_bytes=_VMEM_LIMIT,
            ),
            name=name,
        )(x, x, w, bias)

    return call


def _make_chain_call(specs, H, W, name):
    n = len(specs)
    K0 = specs[0][0]
    C0 = specs[0][1]
    Hp0, Wp0 = H + K0 - 1, W + K0 - 1
    n_scratch = 0 if n == 1 else 2
    Cmid = max(s[2] for s in specs[:-1]) if n > 1 else 0

    def body(x_ref, *refs):
        w_refs = refs[0:2 * n:2]
        b_refs = refs[1:2 * n:2]
        out_ref = refs[2 * n]
        scr = list(refs[2 * n + 1:])

        for s in scr:
            s[...] = jnp.zeros_like(s)

        src_ref = x_ref
        src_is_input = True
        for i, (K, Cin, Cout, relu) in enumerate(specs):
            acc = jnp.zeros((H * W, Cout), jnp.float32)
            for kh in range(K):
                for kw in range(K):
                    if src_is_input:
                        xs = src_ref[0, kh:kh + H, kw:kw + W, :]
                    else:
                        xs = src_ref[kh:kh + H, kw:kw + W, :Cin]
                    xs = xs.reshape(H * W, Cin)
                    acc = acc + jnp.dot(xs, w_refs[i][kh * K + kw],
                                        preferred_element_type=jnp.float32)
            y = acc + b_refs[i][...]
            if relu:
                y = jnp.maximum(y, 0.0)
            if i == n - 1:
                out_ref[0] = y
            else:
                dst = scr[i % 2]
                dst[1:H + 1, 1:W + 1, :Cout] = y.reshape(H, W, Cout)
                src_ref = dst
                src_is_input = False

    Cout_last = specs[-1][2]

    def call(x, weights, biases):
        in_specs = [pl.BlockSpec((1, Hp0, Wp0, C0), lambda b: (b, 0, 0, 0))]
        args = [x]
        for (K, Cin, Cout, relu), w, bias in zip(specs, weights, biases):
            in_specs.append(pl.BlockSpec((K * K, Cin, Cout), lambda b: (0, 0, 0)))
            in_specs.append(pl.BlockSpec((1, Cout), lambda b: (0, 0)))
            args.append(w)
            args.append(bias)
        scratch = []
        if n_scratch:
            scratch = [pltpu.VMEM((H + 2, W + 2, Cmid), jnp.float32)
                       for _ in range(2)]
        B = x.shape[0]
        return pl.pallas_call(
            body,
            out_shape=jax.ShapeDtypeStruct((B, H * W, Cout_last), jnp.float32),
            grid=(B,),
            in_specs=in_specs,
            out_specs=pl.BlockSpec((1, H * W, Cout_last), lambda b: (b, 0, 0)),
            scratch_shapes=scratch,
            compiler_params=pltpu.CompilerParams(
                dimension_semantics=("parallel",),
                vmem_limit_bytes=_VMEM_LIMIT,
            ),
            name=name,
        )(*args)

    return call


def _to_nhwc(x):
    return x.transpose(0, 2, 3, 1)


def _pad_hw(x, pad):
    return jnp.pad(x, ((0, 0), (pad, pad), (pad, pad), (0, 0)))


def _s2d(x):
    B, H, W, C = x.shape
    x = x.reshape(B, H // 2, 2, W // 2, 2, C)
    x = x.transpose(0, 1, 3, 2, 4, 5)
    return x.reshape(B, H // 2, W // 2, 4 * C)


def _w_oihw_to_taps(w):
    Cout, Cin, K, _ = w.shape
    return w.transpose(2, 3, 1, 0).reshape(K * K, Cin, Cout)


def _w_stride2(w):
    Cout, Cin, _, _ = w.shape
    w = w.transpose(2, 3, 1, 0)
    w = jnp.pad(w, ((0, 1), (0, 1), (0, 0), (0, 0)))
    w = w.reshape(2, 2, 2, 2, Cin, Cout)
    w = w.transpose(0, 2, 1, 3, 4, 5)
    return w.reshape(4, 4 * Cin, Cout)


def _up2(x):
    B, H, W, C = x.shape
    x = jnp.broadcast_to(x[:, :, None, :, None, :], (B, H, 2, W, 2, C))
    return x.reshape(B, 2 * H, 2 * W, C)


def _loss_body(meta_ref, gtt_ref, gt_ref, pcls_ref, pbox_ref, pctr_ref, out_ref):
    L = meta_ref.shape[0]
    N = gt_ref.shape[1]

    x = meta_ref[:, 0:1]
    y = meta_ref[:, 1:2]
    stride = meta_ref[:, 2:3]
    lower = meta_ref[:, 3:4]
    upper = meta_ref[:, 4:5]

    gtt = gtt_ref[0]
    x1 = gtt[0:1, :]
    y1 = gtt[1:2, :]
    x2 = gtt[2:3, :]
    y2 = gtt[3:4, :]

    dl = x - x1
    dt = y - y1
    dr = x2 - x
    db = y2 - y
    mind = jnp.minimum(jnp.minimum(dl, dt), jnp.minimum(dr, db))
    maxd = jnp.maximum(jnp.maximum(dl, dt), jnp.maximum(dr, db))
    inside = mind > 0.0
    fit = (maxd > lower) & (maxd < upper)
    areas = (x2 - x1) * (y2 - y1)
    quality = jnp.where(inside & fit, 1e8 - areas, 0.0)

    q = jnp.max(quality, axis=1, keepdims=True)
    nidx = lax.broadcasted_iota(jnp.int32, (L, N), 1)
    sel = jnp.where(quality == q, nidx, N)
    idx = jnp.min(sel, axis=1, keepdims=True)
    onehot = nidx == idx

    mcols = []
    for c in range(5):
        mc = jnp.sum(jnp.where(onehot, gtt[c:c + 1, :], 0.0),
                     axis=1, keepdims=True)
        mcols.append(mc)

    bg = q < 1e-5
    mx1 = jnp.where(bg, -1.0, mcols[0])
    my1 = jnp.where(bg, -1.0, mcols[1])
    mx2 = jnp.where(bg, -1.0, mcols[2])
    my2 = jnp.where(bg, -1.0, mcols[3])
    mcls = jnp.where(bg, -1.0, mcols[4])

    gl = jnp.where(bg, -1.0, (x - mx1) / stride)
    gt_ = jnp.where(bg, -1.0, (y - my1) / stride)
    gr = jnp.where(bg, -1.0, (mx2 - x) / stride)
    gb = jnp.where(bg, -1.0, (my2 - y) / stride)

    fg = mcls >= 0.0
    cls_i = jnp.clip(mcls, 0.0, None).astype(jnp.int32)
    ciota = lax.broadcasted_iota(jnp.int32, (L, _NCLS), 1)
    t = jnp.where((ciota == cls_i) & fg, 1.0, 0.0)
    logits = pcls_ref[0]
    p = 1.0 / (1.0 + jnp.exp(-logits))
    ce = jnp.maximum(logits, 0.0) - logits * t + jnp.log1p(jnp.exp(-jnp.abs(logits)))
    p_t = p * t + (1.0 - p) * (1.0 - t)
    a_t = 0.25 * t + 0.75 * (1.0 - t)
    focal = a_t * ce * (1.0 - p_t) * (1.0 - p_t)
    cls_sum = jnp.sum(jnp.sum(focal, axis=0, keepdims=True),
                      axis=1, keepdims=True)

    pbox = pbox_ref[0]
    gt_d = jnp.concatenate([gl, gt_, gr, gb], axis=1)
    l1 = 0.25 * jnp.abs(pbox - gt_d)
    l1 = jnp.where(gt_d < 0.0, 0.0, l1)
    box_sum = jnp.sum(jnp.sum(l1, axis=0, keepdims=True), axis=1, keepdims=True)

    num = jnp.minimum(gl, gr) * jnp.minimum(gt_, gb)
    den = jnp.maximum(gl, gr) * jnp.maximum(gt_, gb)
    ctr = jnp.sqrt(jnp.clip(num / (den + 1e-8), 0.0, None))
    ctr = jnp.where(gl < 0.0, -1.0, ctr)
    valid = ctr >= 0.0
    tc = jnp.where(valid, ctr, 0.0)
    xl = pctr_ref[0]
    bce = jnp.maximum(xl, 0.0) - xl * tc + jnp.log1p(jnp.exp(-jnp.abs(xl)))
    bce = jnp.where(valid, bce, 0.0)
    ctr_sum = jnp.sum(jnp.sum(bce, axis=0, keepdims=True), axis=1, keepdims=True)

    fg_sum = jnp.sum(jnp.sum(jnp.where(fg, 1.0, 0.0), axis=0, keepdims=True),
                     axis=1, keepdims=True)

    out_ref[0] = jnp.concatenate([cls_sum, box_sum, ctr_sum, fg_sum], axis=1)


def _loss_call(meta, gtt, gt, pcls, pbox, pctr):
    B, L, _ = pcls.shape
    N = gt.shape[1]
    return pl.pallas_call(
        _loss_body,
        out_shape=jax.ShapeDtypeStruct((B, 1, 4), jnp.float32),
        grid=(B,),
        in_specs=[
            pl.BlockSpec((L, 8), lambda b: (0, 0)),
            pl.BlockSpec((1, 5, N), lambda b: (b, 0, 0)),
            pl.BlockSpec((1, N, 5), lambda b: (b, 0, 0)),
            pl.BlockSpec((1, L, _NCLS), lambda b: (b, 0, 0)),
            pl.BlockSpec((1, L, 4), lambda b: (b, 0, 0)),
            pl.BlockSpec((1, L, 1), lambda b: (b, 0, 0)),
        ],
        out_specs=pl.BlockSpec((1, 1, 4), lambda b: (b, 0, 0)),
        compiler_params=pltpu.CompilerParams(
            dimension_semantics=("parallel",),
            vmem_limit_bytes=_VMEM_LIMIT,
        ),
        name="fcos_match_loss",
    )(meta, gtt, gt, pcls, pbox, pctr)


def _build_meta(shapes):
    rows = []
    bounds = [(0.0, _STRIDES[0] * 8.0),
              (_STRIDES[1] * 4.0, _STRIDES[1] * 8.0),
              (_STRIDES[2] * 4.0, float("inf"))]
    for (h, w), s, (lo, hi) in zip(shapes, _STRIDES, bounds):
        ys = (jnp.arange(h, dtype=jnp.float32) + 0.5) * s
        xs = (jnp.arange(w, dtype=jnp.float32) + 0.5) * s
        gy, gx = jnp.meshgrid(ys, xs, indexing="ij")
        L = h * w
        m = jnp.stack([
            gx.ravel(), gy.ravel(),
            jnp.full((L,), float(s), jnp.float32),
            jnp.full((L,), lo, jnp.float32),
            jnp.full((L,), hi, jnp.float32),
            jnp.zeros((L,), jnp.float32),
            jnp.zeros((L,), jnp.float32),
            jnp.zeros((L,), jnp.float32),
        ], axis=-1)
        rows.append(m)
    return jnp.concatenate(rows, axis=0)


def kernel(images, gt_boxes, params):
    B = images.shape[0]

    x = _to_nhwc(images)
    feats = []
    bb_tile = (32, 32, None, None, None)
    for i, (w, b) in enumerate(params["backbone"]):
        xp = _pad_hw(x, 1)
        xs = _s2d(xp)
        H = xs.shape[1] - 1
        W = xs.shape[2] - 1
        C4 = xs.shape[3]
        Cout = w.shape[0]
        if bb_tile[i]:
            bh = bb_tile[i]
            T = H // bh
            pad_rows = T * bh + bh - xs.shape[1]
            xs = jnp.pad(xs, ((0, 0), (0, pad_rows), (0, 0), (0, 0)))
            call = _make_tiled_conv_call(2, C4, Cout, True, H, W, bh, f"bb{i}")
            y = call(xs, _w_stride2(w), b.reshape(1, -1))
        else:
            call = _make_chain_call(((2, C4, Cout, True),), H, W, f"bb{i}")
            y = call(xs, [_w_stride2(w)], [b.reshape(1, -1)])
        x = y.reshape(B, H, W, Cout)
        if i >= 2:
            feats.append(x)

    lats = []
    for f, (w, b) in zip(feats, params["fpn_lat"]):
        H, W, Cin = f.shape[1], f.shape[2], f.shape[3]
        call = _make_chain_call(((1, Cin, 256, False),), H, W, f"lat{H}")
        lats.append(call(f, [_w_oihw_to_taps(w)],
                         [b.reshape(1, -1)]).reshape(B, H, W, 256))

    p5pre = lats[2]
    p4pre = lats[1] + _up2(p5pre)
    p3pre = lats[0] + _up2(p4pre)

    fpn = []
    for pre, (w, b) in zip([p3pre, p4pre, p5pre], params["fpn_out"]):
        H, W = pre.shape[1], pre.shape[2]
        call = _make_chain_call(((3, 256, 256, False),), H, W, f"fpnout{H}")
        fpn.append(call(_pad_hw(pre, 1), [_w_oihw_to_taps(w)],
                        [b.reshape(1, -1)]).reshape(B, H, W, 256))

    stem_cls_w = [_w_oihw_to_taps(w) for (w, _) in params["stem_cls"]]
    stem_cls_b = [b.reshape(1, -1) for (_, b) in params["stem_cls"]]
    stem_box_w = [_w_oihw_to_taps(w) for (w, _) in params["stem_box"]]
    stem_box_b = [b.reshape(1, -1) for (_, b) in params["stem_box"]]
    wc, bc = params["pred_cls"]
    wb, bbx = params["pred_box"]
    wt, bt = params["pred_ctr"]
    pred_cls_w = _w_oihw_to_taps(wc)
    pred_cls_b = bc.reshape(1, -1)
    pred_bc_w = jnp.concatenate([_w_oihw_to_taps(wb), _w_oihw_to_taps(wt)],
                                axis=-1)
    pred_bc_b = jnp.concatenate([bbx, bt]).reshape(1, -1)

    cls_l, box_l, ctr_l = [], [], []
    for f in fpn:
        H, W = f.shape[1], f.shape[2]
        fp = _pad_hw(f, 1)
        cls_specs = tuple([(3, 256, 256, True)] * 4 + [(3, 256, _NCLS, False)])
        box_specs = tuple([(3, 256, 256, True)] * 4 + [(3, 256, 5, False)])
        cls_call = _make_chain_call(cls_specs, H, W, f"head_cls{H}")
        box_call = _make_chain_call(box_specs, H, W, f"head_box{H}")
        pc = cls_call(fp, stem_cls_w + [pred_cls_w], stem_cls_b + [pred_cls_b])
        pbc = box_call(fp, stem_box_w + [pred_bc_w], stem_box_b + [pred_bc_b])
        cls_l.append(pc)
        box_l.append(pbc[..., 0:4])
        ctr_l.append(pbc[..., 4:5])

    p_cls = jnp.concatenate(cls_l, axis=1)
    p_box = jnp.concatenate(box_l, axis=1)
    p_ctr = jnp.concatenate(ctr_l, axis=1)

    meta = _build_meta([(f.shape[1], f.shape[2]) for f in fpn])
    gtt = gt_boxes.transpose(0, 2, 1)
    sums = _loss_call(meta, gtt, gt_boxes, p_cls, p_box, p_ctr)
    tot = jnp.sum(sums[:, 0, :], axis=0)
    norm = jnp.maximum(tot[3], 1.0)
    return tot[0:3] / norm


# speedup vs baseline: 1.0025x; 1.0025x over previous
import jax
import jax.numpy as jnp
from jax import lax
from jax.experimental import pallas as pl
from jax.experimental.pallas import tpu as pltpu

_NCLS = 20
_STRIDES = (8, 16, 32)
_VMEM_LIMIT = 52 * 1024 * 1024


def _make_tiled_conv_call(K, Cin, Cout, relu, H, W, bh, name):
    T = H // bh
    Wp = W + K - 1

    def body(a_ref, b_ref, w_ref, bias_ref, out_ref):
        xin = jnp.concatenate([a_ref[0], b_ref[0][:K - 1]], axis=0)
        acc = jnp.zeros((bh * W, Cout), jnp.float32)
        for kh in range(K):
            for kw in range(K):
                xs = xin[kh:kh + bh, kw:kw + W, :].reshape(bh * W, Cin)
                acc = acc + jnp.dot(xs.astype(jnp.bfloat16), w_ref[kh * K + kw],
                                    preferred_element_type=jnp.float32)
        y = acc + bias_ref[...]
        if relu:
            y = jnp.maximum(y, 0.0)
        out_ref[0] = y

    def call(x, w, bias):
        B = x.shape[0]
        return pl.pallas_call(
            body,
            out_shape=jax.ShapeDtypeStruct((B, H * W, Cout), jnp.float32),
            grid=(B, T),
            in_specs=[
                pl.BlockSpec((1, bh, Wp, Cin), lambda b, t: (b, t, 0, 0)),
                pl.BlockSpec((1, bh, Wp, Cin), lambda b, t: (b, t + 1, 0, 0)),
                pl.BlockSpec((K * K, Cin, Cout), lambda b, t: (0, 0, 0)),
                pl.BlockSpec((1, Cout), lambda b, t: (0, 0)),
            ],
            out_specs=pl.BlockSpec((1, bh * W, Cout), lambda b, t: (b, t, 0)),
            compiler_params=pltpu.CompilerParams(
                dimension_semantics=("parallel", "arbitrary"),
                vmem_limit_bytes=_VMEM_LIMIT,
            ),
            name=name,
        )(x, x, w, bias)

    return call


def _make_chain_call(specs, H, W, name):
    n = len(specs)
    K0 = specs[0][0]
    C0 = specs[0][1]
    Hp0, Wp0 = H + K0 - 1, W + K0 - 1
    n_scratch = 0 if n == 1 else 2
    Cmid = max(s[2] for s in specs[:-1]) if n > 1 else 0

    def body(x_ref, *refs):
        w_refs = refs[0:2 * n:2]
        b_refs = refs[1:2 * n:2]
        out_ref = refs[2 * n]
        scr = list(refs[2 * n + 1:])
        has_k3 = any(s[0] == 3 for s in specs)
        x3_ref = scr[-1] if has_k3 else None

        for s in (scr[:2] if n > 1 else []):
            s[...] = jnp.zeros_like(s)

        src_ref = x_ref
        src_is_input = True
        for i, (K, Cin, Cout, relu) in enumerate(specs):
            acc = jnp.zeros((H * W, Cout), jnp.float32)
            if K == 3 and x3_ref is not None:
                for kh in range(3):
                    for kw in range(3):
                        if src_is_input:
                            xs = src_ref[0, kh:kh + H, kw:kw + W, :]
                        else:
                            xs = src_ref[kh:kh + H, kw:kw + W, :Cin]
                        x3_ref[:, kw * Cin:(kw + 1) * Cin] = (
                            xs.reshape(H * W, Cin).astype(jnp.bfloat16))
                    acc = acc + jnp.dot(x3_ref[...], w_refs[i][kh],
                                        preferred_element_type=jnp.float32)
            else:
                for kh in range(K):
                    for kw in range(K):
                        if src_is_input:
                            xs = src_ref[0, kh:kh + H, kw:kw + W, :]
                        else:
                            xs = src_ref[kh:kh + H, kw:kw + W, :Cin]
                        xs = xs.reshape(H * W, Cin).astype(jnp.bfloat16)
                        acc = acc + jnp.dot(xs, w_refs[i][kh * K + kw],
                                            preferred_element_type=jnp.float32)
            y = acc + b_refs[i][...]
            if relu:
                y = jnp.maximum(y, 0.0)
            if i == n - 1:
                out_ref[0] = y
            else:
                dst = scr[i % 2]
                dst[1:H + 1, 1:W + 1, :Cout] = y.reshape(H, W, Cout)
                src_ref = dst
                src_is_input = False

    Cout_last = specs[-1][2]

    def call(x, weights, biases):
        in_specs = [pl.BlockSpec((1, Hp0, Wp0, C0), lambda b: (b, 0, 0, 0))]
        args = [x]
        for (K, Cin, Cout, relu), w, bias in zip(specs, weights, biases):
            in_specs.append(pl.BlockSpec(w.shape, lambda b: (0, 0, 0)))
            in_specs.append(pl.BlockSpec((1, Cout), lambda b: (0, 0)))
            args.append(w)
            args.append(bias)
        scratch = []
        if n_scratch:
            scratch = [pltpu.VMEM((H + 2, W + 2, Cmid), jnp.float32)
                       for _ in range(2)]
        if any(s[0] == 3 for s in specs):
            k3cin = max(s[1] for s in specs if s[0] == 3)
            scratch = scratch + [pltpu.VMEM((H * W, 3 * k3cin), jnp.bfloat16)]
        B = x.shape[0]
        return pl.pallas_call(
            body,
            out_shape=jax.ShapeDtypeStruct((B, H * W, Cout_last), jnp.float32),
            grid=(B,),
            in_specs=in_specs,
            out_specs=pl.BlockSpec((1, H * W, Cout_last), lambda b: (b, 0, 0)),
            scratch_shapes=scratch,
            compiler_params=pltpu.CompilerParams(
                dimension_semantics=("parallel",),
                vmem_limit_bytes=_VMEM_LIMIT,
            ),
            name=name,
        )(*args)

    return call


def _to_nhwc(x):
    return x.transpose(0, 2, 3, 1)


def _pad_hw(x, pad):
    return jnp.pad(x, ((0, 0), (pad, pad), (pad, pad), (0, 0)))


def _s2d(x):
    B, H, W, C = x.shape
    x = x.reshape(B, H // 2, 2, W // 2, 2, C)
    x = x.transpose(0, 1, 3, 2, 4, 5)
    return x.reshape(B, H // 2, W // 2, 4 * C)


def _w_oihw_to_taps(w):
    Cout, Cin, K, _ = w.shape
    return w.transpose(2, 3, 1, 0).reshape(K * K, Cin, Cout)


def _w3(w):
    Cout, Cin, _, _ = w.shape
    t = _w_oihw_to_taps(w).reshape(3, 3 * Cin, Cout)
    return t.astype(jnp.bfloat16)


def _w_stride2(w):
    Cout, Cin, _, _ = w.shape
    w = w.transpose(2, 3, 1, 0)
    w = jnp.pad(w, ((0, 1), (0, 1), (0, 0), (0, 0)))
    w = w.reshape(2, 2, 2, 2, Cin, Cout)
    w = w.transpose(0, 2, 1, 3, 4, 5)
    return w.reshape(4, 4 * Cin, Cout)


def _up2(x):
    B, H, W, C = x.shape
    x = jnp.broadcast_to(x[:, :, None, :, None, :], (B, H, 2, W, 2, C))
    return x.reshape(B, 2 * H, 2 * W, C)


def _loss_body(meta_ref, gtt_ref, gt_ref, pcls_ref, pbox_ref, pctr_ref, out_ref):
    L = meta_ref.shape[0]
    N = gt_ref.shape[1]

    x = meta_ref[:, 0:1]
    y = meta_ref[:, 1:2]
    stride = meta_ref[:, 2:3]
    lower = meta_ref[:, 3:4]
    upper = meta_ref[:, 4:5]

    gtt = gtt_ref[0]
    x1 = gtt[0:1, :]
    y1 = gtt[1:2, :]
    x2 = gtt[2:3, :]
    y2 = gtt[3:4, :]

    dl = x - x1
    dt = y - y1
    dr = x2 - x
    db = y2 - y
    mind = jnp.minimum(jnp.minimum(dl, dt), jnp.minimum(dr, db))
    maxd = jnp.maximum(jnp.maximum(dl, dt), jnp.maximum(dr, db))
    inside = mind > 0.0
    fit = (maxd > lower) & (maxd < upper)
    areas = (x2 - x1) * (y2 - y1)
    quality = jnp.where(inside & fit, 1e8 - areas, 0.0)

    q = jnp.max(quality, axis=1, keepdims=True)
    nidx = lax.broadcasted_iota(jnp.int32, (L, N), 1)
    sel = jnp.where(quality == q, nidx, N)
    idx = jnp.min(sel, axis=1, keepdims=True)
    onehot = nidx == idx

    mcols = []
    for c in range(5):
        mc = jnp.sum(jnp.where(onehot, gtt[c:c + 1, :], 0.0),
                     axis=1, keepdims=True)
        mcols.append(mc)

    bg = q < 1e-5
    mx1 = jnp.where(bg, -1.0, mcols[0])
    my1 = jnp.where(bg, -1.0, mcols[1])
    mx2 = jnp.where(bg, -1.0, mcols[2])
    my2 = jnp.where(bg, -1.0, mcols[3])
    mcls = jnp.where(bg, -1.0, mcols[4])

    gl = jnp.where(bg, -1.0, (x - mx1) / stride)
    gt_ = jnp.where(bg, -1.0, (y - my1) / stride)
    gr = jnp.where(bg, -1.0, (mx2 - x) / stride)
    gb = jnp.where(bg, -1.0, (my2 - y) / stride)

    fg = mcls >= 0.0
    cls_i = jnp.clip(mcls, 0.0, None).astype(jnp.int32)
    ciota = lax.broadcasted_iota(jnp.int32, (L, _NCLS), 1)
    t = jnp.where((ciota == cls_i) & fg, 1.0, 0.0)
    logits = pcls_ref[0]
    p = 1.0 / (1.0 + jnp.exp(-logits))
    ce = jnp.maximum(logits, 0.0) - logits * t + jnp.log1p(jnp.exp(-jnp.abs(logits)))
    p_t = p * t + (1.0 - p) * (1.0 - t)
    a_t = 0.25 * t + 0.75 * (1.0 - t)
    focal = a_t * ce * (1.0 - p_t) * (1.0 - p_t)
    cls_sum = jnp.sum(jnp.sum(focal, axis=0, keepdims=True),
                      axis=1, keepdims=True)

    pbox = pbox_ref[0]
    gt_d = jnp.concatenate([gl, gt_, gr, gb], axis=1)
    l1 = 0.25 * jnp.abs(pbox - gt_d)
    l1 = jnp.where(gt_d < 0.0, 0.0, l1)
    box_sum = jnp.sum(jnp.sum(l1, axis=0, keepdims=True), axis=1, keepdims=True)

    num = jnp.minimum(gl, gr) * jnp.minimum(gt_, gb)
    den = jnp.maximum(gl, gr) * jnp.maximum(gt_, gb)
    ctr = jnp.sqrt(jnp.clip(num / (den + 1e-8), 0.0, None))
    ctr = jnp.where(gl < 0.0, -1.0, ctr)
    valid = ctr >= 0.0
    tc = jnp.where(valid, ctr, 0.0)
    xl = pctr_ref[0]
    bce = jnp.maximum(xl, 0.0) - xl * tc + jnp.log1p(jnp.exp(-jnp.abs(xl)))
    bce = jnp.where(valid, bce, 0.0)
    ctr_sum = jnp.sum(jnp.sum(bce, axis=0, keepdims=True), axis=1, keepdims=True)

    fg_sum = jnp.sum(jnp.sum(jnp.where(fg, 1.0, 0.0), axis=0, keepdims=True),
                     axis=1, keepdims=True)

    out_ref[0] = jnp.concatenate([cls_sum, box_sum, ctr_sum, fg_sum], axis=1)


def _loss_call(meta, gtt, gt, pcls, pbox, pctr):
    B, L, _ = pcls.shape
    N = gt.shape[1]
    return pl.pallas_call(
        _loss_body,
        out_shape=jax.ShapeDtypeStruct((B, 1, 4), jnp.float32),
        grid=(B,),
        in_specs=[
            pl.BlockSpec((L, 8), lambda b: (0, 0)),
            pl.BlockSpec((1, 5, N), lambda b: (b, 0, 0)),
            pl.BlockSpec((1, N, 5), lambda b: (b, 0, 0)),
            pl.BlockSpec((1, L, _NCLS), lambda b: (b, 0, 0)),
            pl.BlockSpec((1, L, 4), lambda b: (b, 0, 0)),
            pl.BlockSpec((1, L, 1), lambda b: (b, 0, 0)),
        ],
        out_specs=pl.BlockSpec((1, 1, 4), lambda b: (b, 0, 0)),
        compiler_params=pltpu.CompilerParams(
            dimension_semantics=("parallel",),
            vmem_limit_bytes=_VMEM_LIMIT,
        ),
        name="fcos_match_loss",
    )(meta, gtt, gt, pcls, pbox, pctr)


def _build_meta(shapes):
    rows = []
    bounds = [(0.0, _STRIDES[0] * 8.0),
              (_STRIDES[1] * 4.0, _STRIDES[1] * 8.0),
              (_STRIDES[2] * 4.0, float("inf"))]
    for (h, w), s, (lo, hi) in zip(shapes, _STRIDES, bounds):
        ys = (jnp.arange(h, dtype=jnp.float32) + 0.5) * s
        xs = (jnp.arange(w, dtype=jnp.float32) + 0.5) * s
        gy, gx = jnp.meshgrid(ys, xs, indexing="ij")
        L = h * w
        m = jnp.stack([
            gx.ravel(), gy.ravel(),
            jnp.full((L,), float(s), jnp.float32),
            jnp.full((L,), lo, jnp.float32),
            jnp.full((L,), hi, jnp.float32),
            jnp.zeros((L,), jnp.float32),
            jnp.zeros((L,), jnp.float32),
            jnp.zeros((L,), jnp.float32),
        ], axis=-1)
        rows.append(m)
    return jnp.concatenate(rows, axis=0)


def kernel(images, gt_boxes, params):
    B = images.shape[0]

    x = _to_nhwc(images)
    feats = []
    bb_tile = (32, 32, None, None, None)
    for i, (w, b) in enumerate(params["backbone"]):
        xp = _pad_hw(x, 1)
        xs = _s2d(xp)
        H = xs.shape[1] - 1
        W = xs.shape[2] - 1
        C4 = xs.shape[3]
        Cout = w.shape[0]
        if bb_tile[i]:
            bh = bb_tile[i]
            T = H // bh
            pad_rows = T * bh + bh - xs.shape[1]
            xs = jnp.pad(xs, ((0, 0), (0, pad_rows), (0, 0), (0, 0)))
            call = _make_tiled_conv_call(2, C4, Cout, True, H, W, bh, f"bb{i}")
            y = call(xs, _w_stride2(w).astype(jnp.bfloat16), b.reshape(1, -1))
        else:
            call = _make_chain_call(((2, C4, Cout, True),), H, W, f"bb{i}")
            y = call(xs, [_w_stride2(w).astype(jnp.bfloat16)],
                     [b.reshape(1, -1)])
        x = y.reshape(B, H, W, Cout)
        if i >= 2:
            feats.append(x)

    lats = []
    for f, (w, b) in zip(feats, params["fpn_lat"]):
        H, W, Cin = f.shape[1], f.shape[2], f.shape[3]
        call = _make_chain_call(((1, Cin, 256, False),), H, W, f"lat{H}")
        lats.append(call(f, [_w_oihw_to_taps(w).astype(jnp.bfloat16)],
                         [b.reshape(1, -1)]).reshape(B, H, W, 256))

    p5pre = lats[2]
    p4pre = lats[1] + _up2(p5pre)
    p3pre = lats[0] + _up2(p4pre)

    fpn = []
    for pre, (w, b) in zip([p3pre, p4pre, p5pre], params["fpn_out"]):
        H, W = pre.shape[1], pre.shape[2]
        call = _make_chain_call(((3, 256, 256, False),), H, W, f"fpnout{H}")
        fpn.append(call(_pad_hw(pre, 1), [_w3(w)],
                        [b.reshape(1, -1)]).reshape(B, H, W, 256))

    stem_cls_w = [_w3(w) for (w, _) in params["stem_cls"]]
    stem_cls_b = [b.reshape(1, -1) for (_, b) in params["stem_cls"]]
    stem_box_w = [_w3(w) for (w, _) in params["stem_box"]]
    stem_box_b = [b.reshape(1, -1) for (_, b) in params["stem_box"]]
    wc, bc = params["pred_cls"]
    wb, bbx = params["pred_box"]
    wt, bt = params["pred_ctr"]
    pred_cls_w = _w3(wc)
    pred_cls_b = bc.reshape(1, -1)
    pred_bc_w = jnp.concatenate([_w3(wb), _w3(wt)], axis=-1)
    pred_bc_b = jnp.concatenate([bbx, bt]).reshape(1, -1)

    cls_l, box_l, ctr_l = [], [], []
    for f in fpn:
        H, W = f.shape[1], f.shape[2]
        fp = _pad_hw(f, 1)
        cls_specs = tuple([(3, 256, 256, True)] * 4 + [(3, 256, _NCLS, False)])
        box_specs = tuple([(3, 256, 256, True)] * 4 + [(3, 256, 5, False)])
        cls_call = _make_chain_call(cls_specs, H, W, f"head_cls{H}")
        box_call = _make_chain_call(box_specs, H, W, f"head_box{H}")
        pc = cls_call(fp, stem_cls_w + [pred_cls_w], stem_cls_b + [pred_cls_b])
        pbc = box_call(fp, stem_box_w + [pred_bc_w], stem_box_b + [pred_bc_b])
        cls_l.append(pc)
        box_l.append(pbc[..., 0:4])
        ctr_l.append(pbc[..., 4:5])

    p_cls = jnp.concatenate(cls_l, axis=1)
    p_box = jnp.concatenate(box_l, axis=1)
    p_ctr = jnp.concatenate(ctr_l, axis=1)

    meta = _build_meta([(f.shape[1], f.shape[2]) for f in fpn])
    gtt = gt_boxes.transpose(0, 2, 1)
    sums = _loss_call(meta, gtt, gt_boxes, p_cls, p_box, p_ctr)
    tot = jnp.sum(sums[:, 0, :], axis=0)
    norm = jnp.maximum(tot[3], 1.0)
    return tot[0:3] / norm


# speedup vs baseline: 3.4295x; 3.4209x over previous
import jax
import jax.numpy as jnp
from jax import lax
from jax.experimental import pallas as pl
from jax.experimental.pallas import tpu as pltpu

_NCLS = 20
_STRIDES = (8, 16, 32)
_VMEM_LIMIT = 52 * 1024 * 1024


def _make_tiled_conv_call(K, Cin, Cout, relu, H, W, bh, name):
    T = H // bh
    Wp = W + K - 1

    def body(a_ref, b_ref, w_ref, bias_ref, out_ref):
        xin = jnp.concatenate([a_ref[0], b_ref[0][:K - 1]], axis=0)
        acc = jnp.zeros((bh * W, Cout), jnp.float32)
        for kh in range(K):
            for kw in range(K):
                xs = xin[kh:kh + bh, kw:kw + W, :].reshape(bh * W, Cin)
                acc = acc + jnp.dot(xs.astype(jnp.bfloat16), w_ref[kh * K + kw],
                                    preferred_element_type=jnp.float32)
        y = acc + bias_ref[...]
        if relu:
            y = jnp.maximum(y, 0.0)
        out_ref[0] = y

    def call(x, w, bias):
        B = x.shape[0]
        return pl.pallas_call(
            body,
            out_shape=jax.ShapeDtypeStruct((B, H * W, Cout), jnp.float32),
            grid=(B, T),
            in_specs=[
                pl.BlockSpec((1, bh, Wp, Cin), lambda b, t: (b, t, 0, 0)),
                pl.BlockSpec((1, bh, Wp, Cin), lambda b, t: (b, t + 1, 0, 0)),
                pl.BlockSpec((K * K, Cin, Cout), lambda b, t: (0, 0, 0)),
                pl.BlockSpec((1, Cout), lambda b, t: (0, 0)),
            ],
            out_specs=pl.BlockSpec((1, bh * W, Cout), lambda b, t: (b, t, 0)),
            compiler_params=pltpu.CompilerParams(
                dimension_semantics=("parallel", "arbitrary"),
                vmem_limit_bytes=_VMEM_LIMIT,
            ),
            name=name,
        )(x, x, w, bias)

    return call


def _make_chain_call(specs, H, W, name):
    n = len(specs)
    K0 = specs[0][0]
    C0 = specs[0][1]
    Hp0, Wp0 = H + K0 - 1, W + K0 - 1
    n_scratch = 0 if n == 1 else 2
    Cmid = max(s[2] for s in specs[:-1]) if n > 1 else 0

    def body(x_ref, *refs):
        w_refs = refs[0:2 * n:2]
        b_refs = refs[1:2 * n:2]
        out_ref = refs[2 * n]
        scr = list(refs[2 * n + 1:])
        has_k3 = any(s[0] == 3 for s in specs)
        x3_ref = scr[-1] if has_k3 else None

        for s in (scr[:2] if n > 1 else []):
            s[...] = jnp.zeros_like(s)

        src_ref = x_ref
        src_is_input = True
        for i, (K, Cin, Cout, relu) in enumerate(specs):
            acc = jnp.zeros((H * W, Cout), jnp.float32)
            if K == 3 and x3_ref is not None:
                for kh in range(3):
                    for kw in range(3):
                        if src_is_input:
                            xs = src_ref[0, kh:kh + H, kw:kw + W, :]
                        else:
                            xs = src_ref[kh:kh + H, kw:kw + W, :Cin]
                        x3_ref[:, kw * Cin:(kw + 1) * Cin] = (
                            xs.reshape(H * W, Cin).astype(jnp.bfloat16))
                    acc = acc + jnp.dot(x3_ref[...], w_refs[i][kh],
                                        preferred_element_type=jnp.float32)
            else:
                for kh in range(K):
                    for kw in range(K):
                        if src_is_input:
                            xs = src_ref[0, kh:kh + H, kw:kw + W, :]
                        else:
                            xs = src_ref[kh:kh + H, kw:kw + W, :Cin]
                        xs = xs.reshape(H * W, Cin).astype(jnp.bfloat16)
                        acc = acc + jnp.dot(xs, w_refs[i][kh * K + kw],
                                            preferred_element_type=jnp.float32)
            y = acc + b_refs[i][...]
            if relu:
                y = jnp.maximum(y, 0.0)
            if i == n - 1:
                out_ref[0] = y
            else:
                dst = scr[i % 2]
                dst[1:H + 1, 1:W + 1, :Cout] = y.reshape(H, W, Cout)
                src_ref = dst
                src_is_input = False

    Cout_last = specs[-1][2]

    def call(x, weights, biases):
        in_specs = [pl.BlockSpec((1, Hp0, Wp0, C0), lambda b: (b, 0, 0, 0))]
        args = [x]
        for (K, Cin, Cout, relu), w, bias in zip(specs, weights, biases):
            in_specs.append(pl.BlockSpec(w.shape, lambda b: (0, 0, 0)))
            in_specs.append(pl.BlockSpec((1, Cout), lambda b: (0, 0)))
            args.append(w)
            args.append(bias)
        scratch = []
        if n_scratch:
            scratch = [pltpu.VMEM((H + 2, W + 2, Cmid), jnp.float32)
                       for _ in range(2)]
        if any(s[0] == 3 for s in specs):
            k3cin = max(s[1] for s in specs if s[0] == 3)
            scratch = scratch + [pltpu.VMEM((H * W, 3 * k3cin), jnp.bfloat16)]
        B = x.shape[0]
        return pl.pallas_call(
            body,
            out_shape=jax.ShapeDtypeStruct((B, H * W, Cout_last), jnp.float32),
            grid=(B,),
            in_specs=in_specs,
            out_specs=pl.BlockSpec((1, H * W, Cout_last), lambda b: (b, 0, 0)),
            scratch_shapes=scratch,
            compiler_params=pltpu.CompilerParams(
                dimension_semantics=("parallel",),
                vmem_limit_bytes=_VMEM_LIMIT,
            ),
            name=name,
        )(*args)

    return call


_BB_T = 4
_BB_BH3 = 16


def _bb012_body(a_ref, b_ref, w0_ref, b0_ref, w1_ref, b1_ref, w2_ref, b2_ref,
                out_ref, x_sc, p_sc, z_sc, l1_sc, l2_sc):
    t = pl.program_id(1)
    x_sc[:, 0:128, :] = a_ref[0]
    x_sc[:, 128:144, :] = b_ref[0][:, 0:16, :]
    l1_sc[...] = jnp.zeros_like(l1_sc)
    l2_sc[...] = jnp.zeros_like(l2_sc)
    p_sc[27:32, :] = jnp.zeros((5, 512), jnp.float32)

    for q in range(70):
        for ci in range(3):
            for kh in range(3):
                for kw in range(3):
                    p_sc[ci * 9 + kh * 3 + kw, :] = (
                        x_sc[ci, 2 * q + kh + 1, kw:kw + 512])
        z = jax.lax.dot_general(
            p_sc[...].astype(jnp.bfloat16), w0_ref[...],
            (((0,), (0,)), ((), ())),
            preferred_element_type=jnp.float32)
        z_sc[...] = jnp.maximum(z + b0_ref[...], 0.0)
        l1_sc[q, 1:257, 0:32] = z_sc[0:512:2, :]

    @pl.when(t == 0)
    def _():
        l1_sc[0:3, :, :] = jnp.zeros_like(l1_sc[0:3, :, :])

    @pl.when(t == _BB_T - 1)
    def _():
        l1_sc[67:70, :, :] = jnp.zeros_like(l1_sc[67:70, :, :])

    acc1 = jnp.zeros((34 * 128, 64), jnp.float32)
    for kh in range(3):
        for kw in range(3):
            xs = l1_sc[kh:kh + 68:2, kw:kw + 256:2, 0:32]
            xs = xs.reshape(34 * 128, 32).astype(jnp.bfloat16)
            acc1 = acc1 + jnp.dot(xs, w1_ref[kh * 3 + kw],
                                  preferred_element_type=jnp.float32)
    y1 = jnp.maximum(acc1 + b1_ref[...], 0.0).reshape(34, 128, 64)
    l2_sc[0:34, 1:129, 0:64] = y1

    @pl.when(t == 0)
    def _():
        l2_sc[0, :, :] = jnp.zeros_like(l2_sc[0, :, :])

    @pl.when(t == _BB_T - 1)
    def _():
        l2_sc[33, :, :] = jnp.zeros_like(l2_sc[33, :, :])

    acc2 = jnp.zeros((16 * 64, 64), jnp.float32)
    for kh in range(3):
        for kw in range(3):
            xs = l2_sc[kh:kh + 32:2, kw:kw + 128:2, 0:64]
            xs = xs.reshape(16 * 64, 64).astype(jnp.bfloat16)
            acc2 = acc2 + jnp.dot(xs, w2_ref[kh * 3 + kw],
                                  preferred_element_type=jnp.float32)
    out_ref[0] = jnp.maximum(acc2 + b2_ref[...], 0.0)


def _bb012_call(images, w0, b0, w1, b1, w2, b2):
    B = images.shape[0]
    xp = jnp.pad(images, ((0, 0), (0, 0), (8, 120), (1, 1)))
    return pl.pallas_call(
        _bb012_body,
        out_shape=jax.ShapeDtypeStruct((B, 64 * 64, 64), jnp.float32),
        grid=(B, _BB_T),
        in_specs=[
            pl.BlockSpec((1, 3, 128, 514), lambda b, t: (b, 0, t, 0)),
            pl.BlockSpec((1, 3, 128, 514), lambda b, t: (b, 0, t + 1, 0)),
            pl.BlockSpec((32, 32), lambda b, t: (0, 0)),
            pl.BlockSpec((1, 32), lambda b, t: (0, 0)),
            pl.BlockSpec((9, 32, 64), lambda b, t: (0, 0, 0)),
            pl.BlockSpec((1, 64), lambda b, t: (0, 0)),
            pl.BlockSpec((9, 64, 64), lambda b, t: (0, 0, 0)),
            pl.BlockSpec((1, 64), lambda b, t: (0, 0)),
        ],
        out_specs=pl.BlockSpec((1, _BB_BH3 * 64, 64), lambda b, t: (b, t, 0)),
        scratch_shapes=[
            pltpu.VMEM((3, 144, 514), jnp.float32),
            pltpu.VMEM((32, 512), jnp.float32),
            pltpu.VMEM((512, 32), jnp.float32),
            pltpu.VMEM((72, 258, 32), jnp.float32),
            pltpu.VMEM((40, 130, 64), jnp.float32),
        ],
        compiler_params=pltpu.CompilerParams(
            dimension_semantics=("parallel", "arbitrary"),
            vmem_limit_bytes=_VMEM_LIMIT,
        ),
        name="bb012",
    )(xp, xp, w0, b0, w1, b1, w2, b2)


def _to_nhwc(x):
    return x.transpose(0, 2, 3, 1)


def _pad_hw(x, pad):
    return jnp.pad(x, ((0, 0), (pad, pad), (pad, pad), (0, 0)))


def _s2d(x):
    B, H, W, C = x.shape
    x = x.reshape(B, H // 2, 2, W // 2, 2, C)
    x = x.transpose(0, 1, 3, 2, 4, 5)
    return x.reshape(B, H // 2, W // 2, 4 * C)


def _w_oihw_to_taps(w):
    Cout, Cin, K, _ = w.shape
    return w.transpose(2, 3, 1, 0).reshape(K * K, Cin, Cout)


def _w3(w):
    Cout, Cin, _, _ = w.shape
    t = _w_oihw_to_taps(w).reshape(3, 3 * Cin, Cout)
    return t.astype(jnp.bfloat16)


def _w_stride2(w):
    Cout, Cin, _, _ = w.shape
    w = w.transpose(2, 3, 1, 0)
    w = jnp.pad(w, ((0, 1), (0, 1), (0, 0), (0, 0)))
    w = w.reshape(2, 2, 2, 2, Cin, Cout)
    w = w.transpose(0, 2, 1, 3, 4, 5)
    return w.reshape(4, 4 * Cin, Cout)


def _up2(x):
    B, H, W, C = x.shape
    x = jnp.broadcast_to(x[:, :, None, :, None, :], (B, H, 2, W, 2, C))
    return x.reshape(B, 2 * H, 2 * W, C)


def _loss_body(meta_ref, gtt_ref, gt_ref, pcls_ref, pbox_ref, pctr_ref, out_ref):
    L = meta_ref.shape[0]
    N = gt_ref.shape[1]

    x = meta_ref[:, 0:1]
    y = meta_ref[:, 1:2]
    stride = meta_ref[:, 2:3]
    lower = meta_ref[:, 3:4]
    upper = meta_ref[:, 4:5]

    gtt = gtt_ref[0]
    x1 = gtt[0:1, :]
    y1 = gtt[1:2, :]
    x2 = gtt[2:3, :]
    y2 = gtt[3:4, :]

    dl = x - x1
    dt = y - y1
    dr = x2 - x
    db = y2 - y
    mind = jnp.minimum(jnp.minimum(dl, dt), jnp.minimum(dr, db))
    maxd = jnp.maximum(jnp.maximum(dl, dt), jnp.maximum(dr, db))
    inside = mind > 0.0
    fit = (maxd > lower) & (maxd < upper)
    areas = (x2 - x1) * (y2 - y1)
    quality = jnp.where(inside & fit, 1e8 - areas, 0.0)

    q = jnp.max(quality, axis=1, keepdims=True)
    nidx = lax.broadcasted_iota(jnp.int32, (L, N), 1)
    sel = jnp.where(quality == q, nidx, N)
    idx = jnp.min(sel, axis=1, keepdims=True)
    onehot = nidx == idx

    mcols = []
    for c in range(5):
        mc = jnp.sum(jnp.where(onehot, gtt[c:c + 1, :], 0.0),
                     axis=1, keepdims=True)
        mcols.append(mc)

    bg = q < 1e-5
    mx1 = jnp.where(bg, -1.0, mcols[0])
    my1 = jnp.where(bg, -1.0, mcols[1])
    mx2 = jnp.where(bg, -1.0, mcols[2])
    my2 = jnp.where(bg, -1.0, mcols[3])
    mcls = jnp.where(bg, -1.0, mcols[4])

    gl = jnp.where(bg, -1.0, (x - mx1) / stride)
    gt_ = jnp.where(bg, -1.0, (y - my1) / stride)
    gr = jnp.where(bg, -1.0, (mx2 - x) / stride)
    gb = jnp.where(bg, -1.0, (my2 - y) / stride)

    fg = mcls >= 0.0
    cls_i = jnp.clip(mcls, 0.0, None).astype(jnp.int32)
    ciota = lax.broadcasted_iota(jnp.int32, (L, _NCLS), 1)
    t = jnp.where((ciota == cls_i) & fg, 1.0, 0.0)
    logits = pcls_ref[0]
    p = 1.0 / (1.0 + jnp.exp(-logits))
    ce = jnp.maximum(logits, 0.0) - logits * t + jnp.log1p(jnp.exp(-jnp.abs(logits)))
    p_t = p * t + (1.0 - p) * (1.0 - t)
    a_t = 0.25 * t + 0.75 * (1.0 - t)
    focal = a_t * ce * (1.0 - p_t) * (1.0 - p_t)
    cls_sum = jnp.sum(jnp.sum(focal, axis=0, keepdims=True),
                      axis=1, keepdims=True)

    pbox = pbox_ref[0]
    gt_d = jnp.concatenate([gl, gt_, gr, gb], axis=1)
    l1 = 0.25 * jnp.abs(pbox - gt_d)
    l1 = jnp.where(gt_d < 0.0, 0.0, l1)
    box_sum = jnp.sum(jnp.sum(l1, axis=0, keepdims=True), axis=1, keepdims=True)

    num = jnp.minimum(gl, gr) * jnp.minimum(gt_, gb)
    den = jnp.maximum(gl, gr) * jnp.maximum(gt_, gb)
    ctr = jnp.sqrt(jnp.clip(num / (den + 1e-8), 0.0, None))
    ctr = jnp.where(gl < 0.0, -1.0, ctr)
    valid = ctr >= 0.0
    tc = jnp.where(valid, ctr, 0.0)
    xl = pctr_ref[0]
    bce = jnp.maximum(xl, 0.0) - xl * tc + jnp.log1p(jnp.exp(-jnp.abs(xl)))
    bce = jnp.where(valid, bce, 0.0)
    ctr_sum = jnp.sum(jnp.sum(bce, axis=0, keepdims=True), axis=1, keepdims=True)

    fg_sum = jnp.sum(jnp.sum(jnp.where(fg, 1.0, 0.0), axis=0, keepdims=True),
                     axis=1, keepdims=True)

    out_ref[0] = jnp.concatenate([cls_sum, box_sum, ctr_sum, fg_sum], axis=1)


def _loss_call(meta, gtt, gt, pcls, pbox, pctr):
    B, L, _ = pcls.shape
    N = gt.shape[1]
    return pl.pallas_call(
        _loss_body,
        out_shape=jax.ShapeDtypeStruct((B, 1, 4), jnp.float32),
        grid=(B,),
        in_specs=[
            pl.BlockSpec((L, 8), lambda b: (0, 0)),
            pl.BlockSpec((1, 5, N), lambda b: (b, 0, 0)),
            pl.BlockSpec((1, N, 5), lambda b: (b, 0, 0)),
            pl.BlockSpec((1, L, _NCLS), lambda b: (b, 0, 0)),
            pl.BlockSpec((1, L, 4), lambda b: (b, 0, 0)),
            pl.BlockSpec((1, L, 1), lambda b: (b, 0, 0)),
        ],
        out_specs=pl.BlockSpec((1, 1, 4), lambda b: (b, 0, 0)),
        compiler_params=pltpu.CompilerParams(
            dimension_semantics=("parallel",),
            vmem_limit_bytes=_VMEM_LIMIT,
        ),
        name="fcos_match_loss",
    )(meta, gtt, gt, pcls, pbox, pctr)


def _build_meta(shapes):
    rows = []
    bounds = [(0.0, _STRIDES[0] * 8.0),
              (_STRIDES[1] * 4.0, _STRIDES[1] * 8.0),
              (_STRIDES[2] * 4.0, float("inf"))]
    for (h, w), s, (lo, hi) in zip(shapes, _STRIDES, bounds):
        ys = (jnp.arange(h, dtype=jnp.float32) + 0.5) * s
        xs = (jnp.arange(w, dtype=jnp.float32) + 0.5) * s
        gy, gx = jnp.meshgrid(ys, xs, indexing="ij")
        L = h * w
        m = jnp.stack([
            gx.ravel(), gy.ravel(),
            jnp.full((L,), float(s), jnp.float32),
            jnp.full((L,), lo, jnp.float32),
            jnp.full((L,), hi, jnp.float32),
            jnp.zeros((L,), jnp.float32),
            jnp.zeros((L,), jnp.float32),
            jnp.zeros((L,), jnp.float32),
        ], axis=-1)
        rows.append(m)
    return jnp.concatenate(rows, axis=0)


def kernel(images, gt_boxes, params):
    B = images.shape[0]

    bbp = params["backbone"]
    w0 = jnp.pad(bbp[0][0].transpose(1, 2, 3, 0).reshape(27, 32),
                 ((0, 5), (0, 0))).astype(jnp.bfloat16)
    c3f = _bb012_call(
        images, w0, bbp[0][1].reshape(1, -1),
        _w_oihw_to_taps(bbp[1][0]).astype(jnp.bfloat16),
        bbp[1][1].reshape(1, -1),
        _w_oihw_to_taps(bbp[2][0]).astype(jnp.bfloat16),
        bbp[2][1].reshape(1, -1))

    feats = [c3f.reshape(B, 64, 64, 64)]
    x = feats[0]
    for i in (3, 4):
        w, b = bbp[i]
        xp = _pad_hw(x, 1)
        xs = _s2d(xp)
        H = xs.shape[1] - 1
        W = xs.shape[2] - 1
        C4 = xs.shape[3]
        Cout = w.shape[0]
        call = _make_chain_call(((2, C4, Cout, True),), H, W, f"bb{i}")
        y = call(xs, [_w_stride2(w).astype(jnp.bfloat16)],
                 [b.reshape(1, -1)])
        x = y.reshape(B, H, W, Cout)
        feats.append(x)

    lats = []
    for f, (w, b) in zip(feats, params["fpn_lat"]):
        H, W, Cin = f.shape[1], f.shape[2], f.shape[3]
        call = _make_chain_call(((1, Cin, 256, False),), H, W, f"lat{H}")
        lats.append(call(f, [_w_oihw_to_taps(w).astype(jnp.bfloat16)],
                         [b.reshape(1, -1)]).reshape(B, H, W, 256))

    p5pre = lats[2]
    p4pre = lats[1] + _up2(p5pre)
    p3pre = lats[0] + _up2(p4pre)

    fpn = []
    for pre, (w, b) in zip([p3pre, p4pre, p5pre], params["fpn_out"]):
        H, W = pre.shape[1], pre.shape[2]
        call = _make_chain_call(((3, 256, 256, False),), H, W, f"fpnout{H}")
        fpn.append(call(_pad_hw(pre, 1), [_w3(w)],
                        [b.reshape(1, -1)]).reshape(B, H, W, 256))

    stem_cls_w = [_w3(w) for (w, _) in params["stem_cls"]]
    stem_cls_b = [b.reshape(1, -1) for (_, b) in params["stem_cls"]]
    stem_box_w = [_w3(w) for (w, _) in params["stem_box"]]
    stem_box_b = [b.reshape(1, -1) for (_, b) in params["stem_box"]]
    wc, bc = params["pred_cls"]
    wb, bbx = params["pred_box"]
    wt, bt = params["pred_ctr"]
    pred_cls_w = _w3(wc)
    pred_cls_b = bc.reshape(1, -1)
    pred_bc_w = jnp.concatenate([_w3(wb), _w3(wt)], axis=-1)
    pred_bc_b = jnp.concatenate([bbx, bt]).reshape(1, -1)

    cls_l, box_l, ctr_l = [], [], []
    for f in fpn:
        H, W = f.shape[1], f.shape[2]
        fp = _pad_hw(f, 1)
        cls_specs = tuple([(3, 256, 256, True)] * 4 + [(3, 256, _NCLS, False)])
        box_specs = tuple([(3, 256, 256, True)] * 4 + [(3, 256, 5, False)])
        cls_call = _make_chain_call(cls_specs, H, W, f"head_cls{H}")
        box_call = _make_chain_call(box_specs, H, W, f"head_box{H}")
        pc = cls_call(fp, stem_cls_w + [pred_cls_w], stem_cls_b + [pred_cls_b])
        pbc = box_call(fp, stem_box_w + [pred_bc_w], stem_box_b + [pred_bc_b])
        cls_l.append(pc)
        box_l.append(pbc[..., 0:4])
        ctr_l.append(pbc[..., 4:5])

    p_cls = jnp.concatenate(cls_l, axis=1)
    p_box = jnp.concatenate(box_l, axis=1)
    p_ctr = jnp.concatenate(ctr_l, axis=1)

    meta = _build_meta([(f.shape[1], f.shape[2]) for f in fpn])
    gtt = gt_boxes.transpose(0, 2, 1)
    sums = _loss_call(meta, gtt, gt_boxes, p_cls, p_box, p_ctr)
    tot = jnp.sum(sums[:, 0, :], axis=0)
    norm = jnp.maximum(tot[3], 1.0)
    return tot[0:3] / norm


# speedup vs baseline: 3.4751x; 1.0133x over previous
import jax
import jax.numpy as jnp
from jax import lax
from jax.experimental import pallas as pl
from jax.experimental.pallas import tpu as pltpu

_NCLS = 20
_STRIDES = (8, 16, 32)
_VMEM_LIMIT = 52 * 1024 * 1024


def _make_tiled_conv_call(K, Cin, Cout, relu, H, W, bh, name):
    T = H // bh
    Wp = W + K - 1

    def body(a_ref, b_ref, w_ref, bias_ref, out_ref):
        xin = jnp.concatenate([a_ref[0], b_ref[0][:K - 1]], axis=0)
        acc = jnp.zeros((bh * W, Cout), jnp.float32)
        for kh in range(K):
            for kw in range(K):
                xs = xin[kh:kh + bh, kw:kw + W, :].reshape(bh * W, Cin)
                acc = acc + jnp.dot(xs.astype(jnp.bfloat16), w_ref[kh * K + kw],
                                    preferred_element_type=jnp.float32)
        y = acc + bias_ref[...]
        if relu:
            y = jnp.maximum(y, 0.0)
        out_ref[0] = y

    def call(x, w, bias):
        B = x.shape[0]
        return pl.pallas_call(
            body,
            out_shape=jax.ShapeDtypeStruct((B, H * W, Cout), jnp.float32),
            grid=(B, T),
            in_specs=[
                pl.BlockSpec((1, bh, Wp, Cin), lambda b, t: (b, t, 0, 0)),
                pl.BlockSpec((1, bh, Wp, Cin), lambda b, t: (b, t + 1, 0, 0)),
                pl.BlockSpec((K * K, Cin, Cout), lambda b, t: (0, 0, 0)),
                pl.BlockSpec((1, Cout), lambda b, t: (0, 0)),
            ],
            out_specs=pl.BlockSpec((1, bh * W, Cout), lambda b, t: (b, t, 0)),
            compiler_params=pltpu.CompilerParams(
                dimension_semantics=("parallel", "arbitrary"),
                vmem_limit_bytes=_VMEM_LIMIT,
            ),
            name=name,
        )(x, x, w, bias)

    return call


def _make_chain_call(specs, H, W, name, pad_out=False):
    n = len(specs)
    K0 = specs[0][0]
    C0 = specs[0][1]
    Hp0, Wp0 = H + K0 - 1, W + K0 - 1
    n_scratch = 0 if n == 1 else 2
    Cmid = max(s[2] for s in specs[:-1]) if n > 1 else 0

    def body(x_ref, *refs):
        w_refs = refs[0:2 * n:2]
        b_refs = refs[1:2 * n:2]
        out_ref = refs[2 * n]
        scr = list(refs[2 * n + 1:])
        has_k3 = any(s[0] == 3 for s in specs)
        x3_ref = scr[-1] if has_k3 else None

        for s in (scr[:2] if n > 1 else []):
            s[...] = jnp.zeros_like(s)

        src_ref = x_ref
        src_is_input = True
        for i, (K, Cin, Cout, relu) in enumerate(specs):
            acc = jnp.zeros((H * W, Cout), jnp.float32)
            if K == 3 and x3_ref is not None:
                for kh in range(3):
                    for kw in range(3):
                        if src_is_input:
                            xs = src_ref[0, kh:kh + H, kw:kw + W, :]
                        else:
                            xs = src_ref[kh:kh + H, kw:kw + W, :Cin]
                        x3_ref[:, kw * Cin:(kw + 1) * Cin] = (
                            xs.reshape(H * W, Cin).astype(jnp.bfloat16))
                    acc = acc + jnp.dot(x3_ref[...], w_refs[i][kh],
                                        preferred_element_type=jnp.float32)
            else:
                for kh in range(K):
                    for kw in range(K):
                        if src_is_input:
                            xs = src_ref[0, kh:kh + H, kw:kw + W, :]
                        else:
                            xs = src_ref[kh:kh + H, kw:kw + W, :Cin]
                        xs = xs.reshape(H * W, Cin).astype(jnp.bfloat16)
                        acc = acc + jnp.dot(xs, w_refs[i][kh * K + kw],
                                            preferred_element_type=jnp.float32)
            y = acc + b_refs[i][...]
            if relu:
                y = jnp.maximum(y, 0.0)
            if i == n - 1:
                if pad_out:
                    out_ref[0, 0:1, :, :] = jnp.zeros((1, W + 2, Cout),
                                                      jnp.float32)
                    out_ref[0, H + 1:H + 2, :, :] = jnp.zeros((1, W + 2, Cout),
                                                              jnp.float32)
                    out_ref[0, 1:H + 1, 0:1, :] = jnp.zeros((H, 1, Cout),
                                                            jnp.float32)
                    out_ref[0, 1:H + 1, W + 1:W + 2, :] = jnp.zeros(
                        (H, 1, Cout), jnp.float32)
                    out_ref[0, 1:H + 1, 1:W + 1, :] = y.reshape(H, W, Cout)
                else:
                    out_ref[0] = y
            else:
                dst = scr[i % 2]
                dst[1:H + 1, 1:W + 1, :Cout] = y.reshape(H, W, Cout)
                src_ref = dst
                src_is_input = False

    Cout_last = specs[-1][2]

    def call(x, weights, biases):
        in_specs = [pl.BlockSpec((1, Hp0, Wp0, C0), lambda b: (b, 0, 0, 0))]
        args = [x]
        for (K, Cin, Cout, relu), w, bias in zip(specs, weights, biases):
            in_specs.append(pl.BlockSpec(w.shape, lambda b: (0, 0, 0)))
            in_specs.append(pl.BlockSpec((1, Cout), lambda b: (0, 0)))
            args.append(w)
            args.append(bias)
        scratch = []
        if n_scratch:
            scratch = [pltpu.VMEM((H + 2, W + 2, Cmid), jnp.float32)
                       for _ in range(2)]
        if any(s[0] == 3 for s in specs):
            k3cin = max(s[1] for s in specs if s[0] == 3)
            scratch = scratch + [pltpu.VMEM((H * W, 3 * k3cin), jnp.bfloat16)]
        B = x.shape[0]
        if pad_out:
            out_shape = jax.ShapeDtypeStruct((B, H + 2, W + 2, Cout_last),
                                             jnp.float32)
            out_spec = pl.BlockSpec((1, H + 2, W + 2, Cout_last),
                                    lambda b: (b, 0, 0, 0))
        else:
            out_shape = jax.ShapeDtypeStruct((B, H * W, Cout_last),
                                             jnp.float32)
            out_spec = pl.BlockSpec((1, H * W, Cout_last), lambda b: (b, 0, 0))
        return pl.pallas_call(
            body,
            out_shape=out_shape,
            grid=(B,),
            in_specs=in_specs,
            out_specs=out_spec,
            scratch_shapes=scratch,
            compiler_params=pltpu.CompilerParams(
                dimension_semantics=("parallel",),
                vmem_limit_bytes=_VMEM_LIMIT,
            ),
            name=name,
        )(*args)

    return call


_BB_T = 4
_BB_BH3 = 16


def _bb012_body(a_ref, b_ref, w0_ref, b0_ref, w1_ref, b1_ref, w2_ref, b2_ref,
                out_ref, x_sc, p_sc, z_sc, l1_sc, l2_sc):
    t = pl.program_id(1)
    x_sc[:, 0:128, :] = a_ref[0]
    x_sc[:, 128:144, :] = b_ref[0][:, 0:16, :]
    l1_sc[...] = jnp.zeros_like(l1_sc)
    l2_sc[...] = jnp.zeros_like(l2_sc)
    p_sc[27:32, :] = jnp.zeros((5, 512), jnp.float32)

    for q in range(70):
        for ci in range(3):
            for kh in range(3):
                for kw in range(3):
                    p_sc[ci * 9 + kh * 3 + kw, :] = (
                        x_sc[ci, 2 * q + kh + 1, kw:kw + 512])
        z = jax.lax.dot_general(
            p_sc[...].astype(jnp.bfloat16), w0_ref[...],
            (((0,), (0,)), ((), ())),
            preferred_element_type=jnp.float32)
        z_sc[...] = jnp.maximum(z + b0_ref[...], 0.0)
        l1_sc[q, 1:257, 0:32] = z_sc[0:512:2, :]

    @pl.when(t == 0)
    def _():
        l1_sc[0:3, :, :] = jnp.zeros_like(l1_sc[0:3, :, :])

    @pl.when(t == _BB_T - 1)
    def _():
        l1_sc[67:70, :, :] = jnp.zeros_like(l1_sc[67:70, :, :])

    acc1 = jnp.zeros((34 * 128, 64), jnp.float32)
    for kh in range(3):
        for kw in range(3):
            xs = l1_sc[kh:kh + 68:2, kw:kw + 256:2, 0:32]
            xs = xs.reshape(34 * 128, 32).astype(jnp.bfloat16)
            acc1 = acc1 + jnp.dot(xs, w1_ref[kh * 3 + kw],
                                  preferred_element_type=jnp.float32)
    y1 = jnp.maximum(acc1 + b1_ref[...], 0.0).reshape(34, 128, 64)
    l2_sc[0:34, 1:129, 0:64] = y1

    @pl.when(t == 0)
    def _():
        l2_sc[0, :, :] = jnp.zeros_like(l2_sc[0, :, :])

    @pl.when(t == _BB_T - 1)
    def _():
        l2_sc[33, :, :] = jnp.zeros_like(l2_sc[33, :, :])

    acc2 = jnp.zeros((16 * 64, 64), jnp.float32)
    for kh in range(3):
        for kw in range(3):
            xs = l2_sc[kh:kh + 32:2, kw:kw + 128:2, 0:64]
            xs = xs.reshape(16 * 64, 64).astype(jnp.bfloat16)
            acc2 = acc2 + jnp.dot(xs, w2_ref[kh * 3 + kw],
                                  preferred_element_type=jnp.float32)
    out_ref[0] = jnp.maximum(acc2 + b2_ref[...], 0.0)


def _bb012_call(images, w0, b0, w1, b1, w2, b2):
    B = images.shape[0]
    xp = jnp.pad(images, ((0, 0), (0, 0), (8, 120), (1, 1)))
    return pl.pallas_call(
        _bb012_body,
        out_shape=jax.ShapeDtypeStruct((B, 64 * 64, 64), jnp.float32),
        grid=(B, _BB_T),
        in_specs=[
            pl.BlockSpec((1, 3, 128, 514), lambda b, t: (b, 0, t, 0)),
            pl.BlockSpec((1, 3, 128, 514), lambda b, t: (b, 0, t + 1, 0)),
            pl.BlockSpec((32, 32), lambda b, t: (0, 0)),
            pl.BlockSpec((1, 32), lambda b, t: (0, 0)),
            pl.BlockSpec((9, 32, 64), lambda b, t: (0, 0, 0)),
            pl.BlockSpec((1, 64), lambda b, t: (0, 0)),
            pl.BlockSpec((9, 64, 64), lambda b, t: (0, 0, 0)),
            pl.BlockSpec((1, 64), lambda b, t: (0, 0)),
        ],
        out_specs=pl.BlockSpec((1, _BB_BH3 * 64, 64), lambda b, t: (b, t, 0)),
        scratch_shapes=[
            pltpu.VMEM((3, 144, 514), jnp.float32),
            pltpu.VMEM((32, 512), jnp.float32),
            pltpu.VMEM((512, 32), jnp.float32),
            pltpu.VMEM((72, 258, 32), jnp.float32),
            pltpu.VMEM((40, 130, 64), jnp.float32),
        ],
        compiler_params=pltpu.CompilerParams(
            dimension_semantics=("parallel", "arbitrary"),
            vmem_limit_bytes=_VMEM_LIMIT,
        ),
        name="bb012",
    )(xp, xp, w0, b0, w1, b1, w2, b2)


def _to_nhwc(x):
    return x.transpose(0, 2, 3, 1)


def _pad_hw(x, pad):
    return jnp.pad(x, ((0, 0), (pad, pad), (pad, pad), (0, 0)))


def _s2d(x):
    B, H, W, C = x.shape
    x = x.reshape(B, H // 2, 2, W // 2, 2, C)
    x = x.transpose(0, 1, 3, 2, 4, 5)
    return x.reshape(B, H // 2, W // 2, 4 * C)


def _w_oihw_to_taps(w):
    Cout, Cin, K, _ = w.shape
    return w.transpose(2, 3, 1, 0).reshape(K * K, Cin, Cout)


def _w3(w):
    Cout, Cin, _, _ = w.shape
    t = _w_oihw_to_taps(w).reshape(3, 3 * Cin, Cout)
    return t.astype(jnp.bfloat16)


def _w_stride2(w):
    Cout, Cin, _, _ = w.shape
    w = w.transpose(2, 3, 1, 0)
    w = jnp.pad(w, ((0, 1), (0, 1), (0, 0), (0, 0)))
    w = w.reshape(2, 2, 2, 2, Cin, Cout)
    w = w.transpose(0, 2, 1, 3, 4, 5)
    return w.reshape(4, 4 * Cin, Cout)


def _up2(x):
    B, H, W, C = x.shape
    x = jnp.broadcast_to(x[:, :, None, :, None, :], (B, H, 2, W, 2, C))
    return x.reshape(B, 2 * H, 2 * W, C)


def _loss_body(meta_ref, gtt_ref, gt_ref, pcls_ref, pbox_ref, pctr_ref, out_ref):
    L = meta_ref.shape[0]
    N = gt_ref.shape[1]

    x = meta_ref[:, 0:1]
    y = meta_ref[:, 1:2]
    stride = meta_ref[:, 2:3]
    lower = meta_ref[:, 3:4]
    upper = meta_ref[:, 4:5]

    gtt = gtt_ref[0]
    x1 = gtt[0:1, :]
    y1 = gtt[1:2, :]
    x2 = gtt[2:3, :]
    y2 = gtt[3:4, :]

    dl = x - x1
    dt = y - y1
    dr = x2 - x
    db = y2 - y
    mind = jnp.minimum(jnp.minimum(dl, dt), jnp.minimum(dr, db))
    maxd = jnp.maximum(jnp.maximum(dl, dt), jnp.maximum(dr, db))
    inside = mind > 0.0
    fit = (maxd > lower) & (maxd < upper)
    areas = (x2 - x1) * (y2 - y1)
    quality = jnp.where(inside & fit, 1e8 - areas, 0.0)

    q = jnp.max(quality, axis=1, keepdims=True)
    nidx = lax.broadcasted_iota(jnp.int32, (L, N), 1)
    sel = jnp.where(quality == q, nidx, N)
    idx = jnp.min(sel, axis=1, keepdims=True)
    onehot = nidx == idx

    mcols = []
    for c in range(5):
        mc = jnp.sum(jnp.where(onehot, gtt[c:c + 1, :], 0.0),
                     axis=1, keepdims=True)
        mcols.append(mc)

    bg = q < 1e-5
    mx1 = jnp.where(bg, -1.0, mcols[0])
    my1 = jnp.where(bg, -1.0, mcols[1])
    mx2 = jnp.where(bg, -1.0, mcols[2])
    my2 = jnp.where(bg, -1.0, mcols[3])
    mcls = jnp.where(bg, -1.0, mcols[4])

    gl = jnp.where(bg, -1.0, (x - mx1) / stride)
    gt_ = jnp.where(bg, -1.0, (y - my1) / stride)
    gr = jnp.where(bg, -1.0, (mx2 - x) / stride)
    gb = jnp.where(bg, -1.0, (my2 - y) / stride)

    fg = mcls >= 0.0
    cls_i = jnp.clip(mcls, 0.0, None).astype(jnp.int32)
    ciota = lax.broadcasted_iota(jnp.int32, (L, _NCLS), 1)
    t = jnp.where((ciota == cls_i) & fg, 1.0, 0.0)
    logits = pcls_ref[0]
    p = 1.0 / (1.0 + jnp.exp(-logits))
    ce = jnp.maximum(logits, 0.0) - logits * t + jnp.log1p(jnp.exp(-jnp.abs(logits)))
    p_t = p * t + (1.0 - p) * (1.0 - t)
    a_t = 0.25 * t + 0.75 * (1.0 - t)
    focal = a_t * ce * (1.0 - p_t) * (1.0 - p_t)
    cls_sum = jnp.sum(jnp.sum(focal, axis=0, keepdims=True),
                      axis=1, keepdims=True)

    pbox = pbox_ref[0]
    gt_d = jnp.concatenate([gl, gt_, gr, gb], axis=1)
    l1 = 0.25 * jnp.abs(pbox - gt_d)
    l1 = jnp.where(gt_d < 0.0, 0.0, l1)
    box_sum = jnp.sum(jnp.sum(l1, axis=0, keepdims=True), axis=1, keepdims=True)

    num = jnp.minimum(gl, gr) * jnp.minimum(gt_, gb)
    den = jnp.maximum(gl, gr) * jnp.maximum(gt_, gb)
    ctr = jnp.sqrt(jnp.clip(num / (den + 1e-8), 0.0, None))
    ctr = jnp.where(gl < 0.0, -1.0, ctr)
    valid = ctr >= 0.0
    tc = jnp.where(valid, ctr, 0.0)
    xl = pctr_ref[0]
    bce = jnp.maximum(xl, 0.0) - xl * tc + jnp.log1p(jnp.exp(-jnp.abs(xl)))
    bce = jnp.where(valid, bce, 0.0)
    ctr_sum = jnp.sum(jnp.sum(bce, axis=0, keepdims=True), axis=1, keepdims=True)

    fg_sum = jnp.sum(jnp.sum(jnp.where(fg, 1.0, 0.0), axis=0, keepdims=True),
                     axis=1, keepdims=True)

    out_ref[0] = jnp.concatenate([cls_sum, box_sum, ctr_sum, fg_sum], axis=1)


def _loss_call(meta, gtt, gt, pcls, pbox, pctr):
    B, L, _ = pcls.shape
    N = gt.shape[1]
    return pl.pallas_call(
        _loss_body,
        out_shape=jax.ShapeDtypeStruct((B, 1, 4), jnp.float32),
        grid=(B,),
        in_specs=[
            pl.BlockSpec((L, 8), lambda b: (0, 0)),
            pl.BlockSpec((1, 5, N), lambda b: (b, 0, 0)),
            pl.BlockSpec((1, N, 5), lambda b: (b, 0, 0)),
            pl.BlockSpec((1, L, _NCLS), lambda b: (b, 0, 0)),
            pl.BlockSpec((1, L, 4), lambda b: (b, 0, 0)),
            pl.BlockSpec((1, L, 1), lambda b: (b, 0, 0)),
        ],
        out_specs=pl.BlockSpec((1, 1, 4), lambda b: (b, 0, 0)),
        compiler_params=pltpu.CompilerParams(
            dimension_semantics=("parallel",),
            vmem_limit_bytes=_VMEM_LIMIT,
        ),
        name="fcos_match_loss",
    )(meta, gtt, gt, pcls, pbox, pctr)


def _build_meta(shapes):
    rows = []
    bounds = [(0.0, _STRIDES[0] * 8.0),
              (_STRIDES[1] * 4.0, _STRIDES[1] * 8.0),
              (_STRIDES[2] * 4.0, float("inf"))]
    for (h, w), s, (lo, hi) in zip(shapes, _STRIDES, bounds):
        ys = (jnp.arange(h, dtype=jnp.float32) + 0.5) * s
        xs = (jnp.arange(w, dtype=jnp.float32) + 0.5) * s
        gy, gx = jnp.meshgrid(ys, xs, indexing="ij")
        L = h * w
        m = jnp.stack([
            gx.ravel(), gy.ravel(),
            jnp.full((L,), float(s), jnp.float32),
            jnp.full((L,), lo, jnp.float32),
            jnp.full((L,), hi, jnp.float32),
            jnp.zeros((L,), jnp.float32),
            jnp.zeros((L,), jnp.float32),
            jnp.zeros((L,), jnp.float32),
        ], axis=-1)
        rows.append(m)
    return jnp.concatenate(rows, axis=0)


def kernel(images, gt_boxes, params):
    B = images.shape[0]

    bbp = params["backbone"]
    w0 = jnp.pad(bbp[0][0].transpose(1, 2, 3, 0).reshape(27, 32),
                 ((0, 5), (0, 0))).astype(jnp.bfloat16)
    c3f = _bb012_call(
        images, w0, bbp[0][1].reshape(1, -1),
        _w_oihw_to_taps(bbp[1][0]).astype(jnp.bfloat16),
        bbp[1][1].reshape(1, -1),
        _w_oihw_to_taps(bbp[2][0]).astype(jnp.bfloat16),
        bbp[2][1].reshape(1, -1))

    feats = [c3f.reshape(B, 64, 64, 64)]
    x = feats[0]
    for i in (3, 4):
        w, b = bbp[i]
        xp = _pad_hw(x, 1)
        xs = _s2d(xp)
        H = xs.shape[1] - 1
        W = xs.shape[2] - 1
        C4 = xs.shape[3]
        Cout = w.shape[0]
        call = _make_chain_call(((2, C4, Cout, True),), H, W, f"bb{i}")
        y = call(xs, [_w_stride2(w).astype(jnp.bfloat16)],
                 [b.reshape(1, -1)])
        x = y.reshape(B, H, W, Cout)
        feats.append(x)

    lats = []
    for f, (w, b) in zip(feats, params["fpn_lat"]):
        H, W, Cin = f.shape[1], f.shape[2], f.shape[3]
        call = _make_chain_call(((1, Cin, 256, False),), H, W, f"lat{H}")
        lats.append(call(f, [_w_oihw_to_taps(w).astype(jnp.bfloat16)],
                         [b.reshape(1, -1)]).reshape(B, H, W, 256))

    p5pre = lats[2]
    p4pre = lats[1] + _up2(p5pre)
    p3pre = lats[0] + _up2(p4pre)

    fpn = []
    for pre, (w, b) in zip([p3pre, p4pre, p5pre], params["fpn_out"]):
        H, W = pre.shape[1], pre.shape[2]
        call = _make_chain_call(((3, 256, 256, False),), H, W, f"fpnout{H}",
                                pad_out=True)
        fpn.append(call(_pad_hw(pre, 1), [_w3(w)],
                        [b.reshape(1, -1)]))

    stem_cls_w = [_w3(w) for (w, _) in params["stem_cls"]]
    stem_cls_b = [b.reshape(1, -1) for (_, b) in params["stem_cls"]]
    stem_box_w = [_w3(w) for (w, _) in params["stem_box"]]
    stem_box_b = [b.reshape(1, -1) for (_, b) in params["stem_box"]]
    wc, bc = params["pred_cls"]
    wb, bbx = params["pred_box"]
    wt, bt = params["pred_ctr"]
    pred_cls_w = _w3(wc)
    pred_cls_b = bc.reshape(1, -1)
    pred_bc_w = jnp.concatenate([_w3(wb), _w3(wt)], axis=-1)
    pred_bc_b = jnp.concatenate([bbx, bt]).reshape(1, -1)

    cls_l, box_l, ctr_l = [], [], []
    for fp in fpn:
        H, W = fp.shape[1] - 2, fp.shape[2] - 2
        cls_specs = tuple([(3, 256, 256, True)] * 4 + [(3, 256, _NCLS, False)])
        box_specs = tuple([(3, 256, 256, True)] * 4 + [(3, 256, 5, False)])
        cls_call = _make_chain_call(cls_specs, H, W, f"head_cls{H}")
        box_call = _make_chain_call(box_specs, H, W, f"head_box{H}")
        pc = cls_call(fp, stem_cls_w + [pred_cls_w], stem_cls_b + [pred_cls_b])
        pbc = box_call(fp, stem_box_w + [pred_bc_w], stem_box_b + [pred_bc_b])
        cls_l.append(pc)
        box_l.append(pbc[..., 0:4])
        ctr_l.append(pbc[..., 4:5])

    p_cls = jnp.concatenate(cls_l, axis=1)
    p_box = jnp.concatenate(box_l, axis=1)
    p_ctr = jnp.concatenate(ctr_l, axis=1)

    meta = _build_meta([(f.shape[1] - 2, f.shape[2] - 2) for f in fpn])
    gtt = gt_boxes.transpose(0, 2, 1)
    sums = _loss_call(meta, gtt, gt_boxes, p_cls, p_box, p_ctr)
    tot = jnp.sum(sums[:, 0, :], axis=0)
    norm = jnp.maximum(tot[3], 1.0)
    return tot[0:3] / norm


# speedup vs baseline: 3.5135x; 1.0110x over previous
import jax
import jax.numpy as jnp
from jax import lax
from jax.experimental import pallas as pl
from jax.experimental.pallas import tpu as pltpu

_NCLS = 20
_STRIDES = (8, 16, 32)
_VMEM_LIMIT = 52 * 1024 * 1024


def _make_tiled_conv_call(K, Cin, Cout, relu, H, W, bh, name):
    T = H // bh
    Wp = W + K - 1

    def body(a_ref, b_ref, w_ref, bias_ref, out_ref):
        xin = jnp.concatenate([a_ref[0], b_ref[0][:K - 1]], axis=0)
        acc = jnp.zeros((bh * W, Cout), jnp.float32)
        for kh in range(K):
            for kw in range(K):
                xs = xin[kh:kh + bh, kw:kw + W, :].reshape(bh * W, Cin)
                acc = acc + jnp.dot(xs.astype(jnp.bfloat16), w_ref[kh * K + kw],
                                    preferred_element_type=jnp.float32)
        y = acc + bias_ref[...]
        if relu:
            y = jnp.maximum(y, 0.0)
        out_ref[0] = y

    def call(x, w, bias):
        B = x.shape[0]
        return pl.pallas_call(
            body,
            out_shape=jax.ShapeDtypeStruct((B, H * W, Cout), jnp.float32),
            grid=(B, T),
            in_specs=[
                pl.BlockSpec((1, bh, Wp, Cin), lambda b, t: (b, t, 0, 0)),
                pl.BlockSpec((1, bh, Wp, Cin), lambda b, t: (b, t + 1, 0, 0)),
                pl.BlockSpec((K * K, Cin, Cout), lambda b, t: (0, 0, 0)),
                pl.BlockSpec((1, Cout), lambda b, t: (0, 0)),
            ],
            out_specs=pl.BlockSpec((1, bh * W, Cout), lambda b, t: (b, t, 0)),
            compiler_params=pltpu.CompilerParams(
                dimension_semantics=("parallel", "arbitrary"),
                vmem_limit_bytes=_VMEM_LIMIT,
            ),
            name=name,
        )(x, x, w, bias)

    return call


def _make_chain_call(specs, H, W, name, pad_out=False):
    n = len(specs)
    K0 = specs[0][0]
    C0 = specs[0][1]
    Hp0, Wp0 = H + K0 - 1, W + K0 - 1
    n_scratch = 0 if n == 1 else 2
    Cmid = max(s[2] for s in specs[:-1]) if n > 1 else 0

    def body(x_ref, *refs):
        w_refs = refs[0:2 * n:2]
        b_refs = refs[1:2 * n:2]
        out_ref = refs[2 * n]
        scr = list(refs[2 * n + 1:])
        has_k3 = any(s[0] == 3 for s in specs)
        x3_ref = scr[-1] if has_k3 else None

        for s in (scr[:2] if n > 1 else []):
            s[...] = jnp.zeros_like(s)

        src_ref = x_ref
        src_is_input = True
        for i, (K, Cin, Cout, relu) in enumerate(specs):
            acc = jnp.zeros((H * W, Cout), jnp.float32)
            if K == 3 and x3_ref is not None:
                for kh in range(3):
                    for kw in range(3):
                        if src_is_input:
                            xs = src_ref[0, kh:kh + H, kw:kw + W, :]
                        else:
                            xs = src_ref[kh:kh + H, kw:kw + W, :Cin]
                        x3_ref[:, kw * Cin:(kw + 1) * Cin] = (
                            xs.reshape(H * W, Cin).astype(jnp.bfloat16))
                    acc = acc + jnp.dot(x3_ref[...], w_refs[i][kh],
                                        preferred_element_type=jnp.float32)
            else:
                for kh in range(K):
                    for kw in range(K):
                        if src_is_input:
                            xs = src_ref[0, kh:kh + H, kw:kw + W, :]
                        else:
                            xs = src_ref[kh:kh + H, kw:kw + W, :Cin]
                        xs = xs.reshape(H * W, Cin).astype(jnp.bfloat16)
                        acc = acc + jnp.dot(xs, w_refs[i][kh * K + kw],
                                            preferred_element_type=jnp.float32)
            y = acc + b_refs[i][...]
            if relu:
                y = jnp.maximum(y, 0.0)
            if i == n - 1:
                if pad_out:
                    out_ref[0, 0:1, :, :] = jnp.zeros((1, W + 2, Cout),
                                                      jnp.float32)
                    out_ref[0, H + 1:H + 2, :, :] = jnp.zeros((1, W + 2, Cout),
                                                              jnp.float32)
                    out_ref[0, 1:H + 1, 0:1, :] = jnp.zeros((H, 1, Cout),
                                                            jnp.float32)
                    out_ref[0, 1:H + 1, W + 1:W + 2, :] = jnp.zeros(
                        (H, 1, Cout), jnp.float32)
                    out_ref[0, 1:H + 1, 1:W + 1, :] = y.reshape(H, W, Cout)
                else:
                    out_ref[0] = y
            else:
                dst = scr[i % 2]
                dst[1:H + 1, 1:W + 1, :Cout] = y.reshape(H, W, Cout)
                src_ref = dst
                src_is_input = False

    Cout_last = specs[-1][2]

    def call(x, weights, biases):
        in_specs = [pl.BlockSpec((1, Hp0, Wp0, C0), lambda b: (b, 0, 0, 0))]
        args = [x]
        for (K, Cin, Cout, relu), w, bias in zip(specs, weights, biases):
            in_specs.append(pl.BlockSpec(w.shape, lambda b: (0, 0, 0)))
            in_specs.append(pl.BlockSpec((1, Cout), lambda b: (0, 0)))
            args.append(w)
            args.append(bias)
        scratch = []
        if n_scratch:
            scratch = [pltpu.VMEM((H + 2, W + 2, Cmid), jnp.float32)
                       for _ in range(2)]
        if any(s[0] == 3 for s in specs):
            k3cin = max(s[1] for s in specs if s[0] == 3)
            scratch = scratch + [pltpu.VMEM((H * W, 3 * k3cin), jnp.bfloat16)]
        B = x.shape[0]
        if pad_out:
            out_shape = jax.ShapeDtypeStruct((B, H + 2, W + 2, Cout_last),
                                             jnp.float32)
            out_spec = pl.BlockSpec((1, H + 2, W + 2, Cout_last),
                                    lambda b: (b, 0, 0, 0))
        else:
            out_shape = jax.ShapeDtypeStruct((B, H * W, Cout_last),
                                             jnp.float32)
            out_spec = pl.BlockSpec((1, H * W, Cout_last), lambda b: (b, 0, 0))
        return pl.pallas_call(
            body,
            out_shape=out_shape,
            grid=(B,),
            in_specs=in_specs,
            out_specs=out_spec,
            scratch_shapes=scratch,
            compiler_params=pltpu.CompilerParams(
                dimension_semantics=("parallel",),
                vmem_limit_bytes=_VMEM_LIMIT,
            ),
            name=name,
        )(*args)

    return call


def _make_head_call(H, W, name):
    n_stem = 4

    def body(x_ref, ws_ref, bs_ref, wp_ref, bp_ref, out_ref, s0, s1, x3_ref):
        s0[...] = jnp.zeros_like(s0)
        s1[...] = jnp.zeros_like(s1)
        scr = [s0, s1]
        src_ref = x_ref
        src_is_input = True
        for i in range(n_stem + 1):
            last = i == n_stem
            Cout = 20 if last else 256
            acc = jnp.zeros((H * W, Cout), jnp.float32)
            for kh in range(3):
                for kw in range(3):
                    if src_is_input:
                        xs = src_ref[0, kh:kh + H, kw:kw + W, :]
                    else:
                        xs = src_ref[kh:kh + H, kw:kw + W, :]
                    x3_ref[:, kw * 256:(kw + 1) * 256] = (
                        xs.reshape(H * W, 256).astype(jnp.bfloat16))
                w = wp_ref[0, kh] if last else ws_ref[0, i, kh]
                acc = acc + jnp.dot(x3_ref[...], w,
                                    preferred_element_type=jnp.float32)
            if last:
                out_ref[0, 0] = acc + bp_ref[0]
            else:
                y = jnp.maximum(acc + bs_ref[0, i], 0.0)
                dst = scr[i % 2]
                dst[1:H + 1, 1:W + 1, :] = y.reshape(H, W, 256)
                src_ref = dst
                src_is_input = False

    def call(fp, wstem, bstem, wpred, bpred):
        B = fp.shape[0]
        return pl.pallas_call(
            body,
            out_shape=jax.ShapeDtypeStruct((B, 2, H * W, 20), jnp.float32),
            grid=(B, 2),
            in_specs=[
                pl.BlockSpec((1, H + 2, W + 2, 256), lambda b, r: (b, 0, 0, 0)),
                pl.BlockSpec((1, 4, 3, 768, 256), lambda b, r: (r, 0, 0, 0, 0)),
                pl.BlockSpec((1, 4, 1, 256), lambda b, r: (r, 0, 0, 0)),
                pl.BlockSpec((1, 3, 768, 20), lambda b, r: (r, 0, 0, 0)),
                pl.BlockSpec((1, 1, 20), lambda b, r: (r, 0, 0)),
            ],
            out_specs=pl.BlockSpec((1, 1, H * W, 20), lambda b, r: (b, r, 0, 0)),
            scratch_shapes=[
                pltpu.VMEM((H + 2, W + 2, 256), jnp.float32),
                pltpu.VMEM((H + 2, W + 2, 256), jnp.float32),
                pltpu.VMEM((H * W, 768), jnp.bfloat16),
            ],
            compiler_params=pltpu.CompilerParams(
                dimension_semantics=("parallel", "arbitrary"),
                vmem_limit_bytes=_VMEM_LIMIT,
            ),
            name=name,
        )(fp, wstem, bstem, wpred, bpred)

    return call


_BB_T = 4
_BB_BH3 = 16


def _bb012_body(a_ref, b_ref, w0_ref, b0_ref, w1_ref, b1_ref, w2_ref, b2_ref,
                out_ref, x_sc, p_sc, z_sc, l1_sc, l2_sc):
    t = pl.program_id(1)
    x_sc[:, 0:128, :] = a_ref[0]
    x_sc[:, 128:144, :] = b_ref[0][:, 0:16, :]
    l1_sc[...] = jnp.zeros_like(l1_sc)
    l2_sc[...] = jnp.zeros_like(l2_sc)
    p_sc[27:32, :] = jnp.zeros((5, 512), jnp.float32)

    for q in range(70):
        for ci in range(3):
            for kh in range(3):
                for kw in range(3):
                    p_sc[ci * 9 + kh * 3 + kw, :] = (
                        x_sc[ci, 2 * q + kh + 1, kw:kw + 512])
        z = jax.lax.dot_general(
            p_sc[...].astype(jnp.bfloat16), w0_ref[...],
            (((0,), (0,)), ((), ())),
            preferred_element_type=jnp.float32)
        z_sc[...] = jnp.maximum(z + b0_ref[...], 0.0)
        l1_sc[q, 1:257, 0:32] = z_sc[0:512:2, :]

    @pl.when(t == 0)
    def _():
        l1_sc[0:3, :, :] = jnp.zeros_like(l1_sc[0:3, :, :])

    @pl.when(t == _BB_T - 1)
    def _():
        l1_sc[67:70, :, :] = jnp.zeros_like(l1_sc[67:70, :, :])

    acc1 = jnp.zeros((34 * 128, 64), jnp.float32)
    for kh in range(3):
        for kw in range(3):
            xs = l1_sc[kh:kh + 68:2, kw:kw + 256:2, 0:32]
            xs = xs.reshape(34 * 128, 32).astype(jnp.bfloat16)
            acc1 = acc1 + jnp.dot(xs, w1_ref[kh * 3 + kw],
                                  preferred_element_type=jnp.float32)
    y1 = jnp.maximum(acc1 + b1_ref[...], 0.0).reshape(34, 128, 64)
    l2_sc[0:34, 1:129, 0:64] = y1

    @pl.when(t == 0)
    def _():
        l2_sc[0, :, :] = jnp.zeros_like(l2_sc[0, :, :])

    @pl.when(t == _BB_T - 1)
    def _():
        l2_sc[33, :, :] = jnp.zeros_like(l2_sc[33, :, :])

    acc2 = jnp.zeros((16 * 64, 64), jnp.float32)
    for kh in range(3):
        for kw in range(3):
            xs = l2_sc[kh:kh + 32:2, kw:kw + 128:2, 0:64]
            xs = xs.reshape(16 * 64, 64).astype(jnp.bfloat16)
            acc2 = acc2 + jnp.dot(xs, w2_ref[kh * 3 + kw],
                                  preferred_element_type=jnp.float32)
    out_ref[0] = jnp.maximum(acc2 + b2_ref[...], 0.0)


def _bb012_call(images, w0, b0, w1, b1, w2, b2):
    B = images.shape[0]
    xp = jnp.pad(images, ((0, 0), (0, 0), (8, 120), (1, 1)))
    return pl.pallas_call(
        _bb012_body,
        out_shape=jax.ShapeDtypeStruct((B, 64 * 64, 64), jnp.float32),
        grid=(B, _BB_T),
        in_specs=[
            pl.BlockSpec((1, 3, 128, 514), lambda b, t: (b, 0, t, 0)),
            pl.BlockSpec((1, 3, 128, 514), lambda b, t: (b, 0, t + 1, 0)),
            pl.BlockSpec((32, 32), lambda b, t: (0, 0)),
            pl.BlockSpec((1, 32), lambda b, t: (0, 0)),
            pl.BlockSpec((9, 32, 64), lambda b, t: (0, 0, 0)),
            pl.BlockSpec((1, 64), lambda b, t: (0, 0)),
            pl.BlockSpec((9, 64, 64), lambda b, t: (0, 0, 0)),
            pl.BlockSpec((1, 64), lambda b, t: (0, 0)),
        ],
        out_specs=pl.BlockSpec((1, _BB_BH3 * 64, 64), lambda b, t: (b, t, 0)),
        scratch_shapes=[
            pltpu.VMEM((3, 144, 514), jnp.float32),
            pltpu.VMEM((32, 512), jnp.float32),
            pltpu.VMEM((512, 32), jnp.float32),
            pltpu.VMEM((72, 258, 32), jnp.float32),
            pltpu.VMEM((40, 130, 64), jnp.float32),
        ],
        compiler_params=pltpu.CompilerParams(
            dimension_semantics=("parallel", "arbitrary"),
            vmem_limit_bytes=_VMEM_LIMIT,
        ),
        name="bb012",
    )(xp, xp, w0, b0, w1, b1, w2, b2)


def _to_nhwc(x):
    return x.transpose(0, 2, 3, 1)


def _pad_hw(x, pad):
    return jnp.pad(x, ((0, 0), (pad, pad), (pad, pad), (0, 0)))


def _s2d(x):
    B, H, W, C = x.shape
    x = x.reshape(B, H // 2, 2, W // 2, 2, C)
    x = x.transpose(0, 1, 3, 2, 4, 5)
    return x.reshape(B, H // 2, W // 2, 4 * C)


def _w_oihw_to_taps(w):
    Cout, Cin, K, _ = w.shape
    return w.transpose(2, 3, 1, 0).reshape(K * K, Cin, Cout)


def _w3(w):
    Cout, Cin, _, _ = w.shape
    t = _w_oihw_to_taps(w).reshape(3, 3 * Cin, Cout)
    return t.astype(jnp.bfloat16)


def _w_stride2(w):
    Cout, Cin, _, _ = w.shape
    w = w.transpose(2, 3, 1, 0)
    w = jnp.pad(w, ((0, 1), (0, 1), (0, 0), (0, 0)))
    w = w.reshape(2, 2, 2, 2, Cin, Cout)
    w = w.transpose(0, 2, 1, 3, 4, 5)
    return w.reshape(4, 4 * Cin, Cout)


def _up2(x):
    B, H, W, C = x.shape
    x = jnp.broadcast_to(x[:, :, None, :, None, :], (B, H, 2, W, 2, C))
    return x.reshape(B, 2 * H, 2 * W, C)


def _loss_body(meta_ref, gtt_ref, gt_ref, pcls_ref, pbox_ref, pctr_ref, out_ref):
    L = meta_ref.shape[0]
    N = gt_ref.shape[1]

    x = meta_ref[:, 0:1]
    y = meta_ref[:, 1:2]
    stride = meta_ref[:, 2:3]
    lower = meta_ref[:, 3:4]
    upper = meta_ref[:, 4:5]

    gtt = gtt_ref[0]
    x1 = gtt[0:1, :]
    y1 = gtt[1:2, :]
    x2 = gtt[2:3, :]
    y2 = gtt[3:4, :]

    dl = x - x1
    dt = y - y1
    dr = x2 - x
    db = y2 - y
    mind = jnp.minimum(jnp.minimum(dl, dt), jnp.minimum(dr, db))
    maxd = jnp.maximum(jnp.maximum(dl, dt), jnp.maximum(dr, db))
    inside = mind > 0.0
    fit = (maxd > lower) & (maxd < upper)
    areas = (x2 - x1) * (y2 - y1)
    quality = jnp.where(inside & fit, 1e8 - areas, 0.0)

    q = jnp.max(quality, axis=1, keepdims=True)
    nidx = lax.broadcasted_iota(jnp.int32, (L, N), 1)
    sel = jnp.where(quality == q, nidx, N)
    idx = jnp.min(sel, axis=1, keepdims=True)
    onehot = nidx == idx

    mcols = []
    for c in range(5):
        mc = jnp.sum(jnp.where(onehot, gtt[c:c + 1, :], 0.0),
                     axis=1, keepdims=True)
        mcols.append(mc)

    bg = q < 1e-5
    mx1 = jnp.where(bg, -1.0, mcols[0])
    my1 = jnp.where(bg, -1.0, mcols[1])
    mx2 = jnp.where(bg, -1.0, mcols[2])
    my2 = jnp.where(bg, -1.0, mcols[3])
    mcls = jnp.where(bg, -1.0, mcols[4])

    gl = jnp.where(bg, -1.0, (x - mx1) / stride)
    gt_ = jnp.where(bg, -1.0, (y - my1) / stride)
    gr = jnp.where(bg, -1.0, (mx2 - x) / stride)
    gb = jnp.where(bg, -1.0, (my2 - y) / stride)

    fg = mcls >= 0.0
    cls_i = jnp.clip(mcls, 0.0, None).astype(jnp.int32)
    ciota = lax.broadcasted_iota(jnp.int32, (L, _NCLS), 1)
    t = jnp.where((ciota == cls_i) & fg, 1.0, 0.0)
    logits = pcls_ref[0]
    p = 1.0 / (1.0 + jnp.exp(-logits))
    ce = jnp.maximum(logits, 0.0) - logits * t + jnp.log1p(jnp.exp(-jnp.abs(logits)))
    p_t = p * t + (1.0 - p) * (1.0 - t)
    a_t = 0.25 * t + 0.75 * (1.0 - t)
    focal = a_t * ce * (1.0 - p_t) * (1.0 - p_t)
    cls_sum = jnp.sum(jnp.sum(focal, axis=0, keepdims=True),
                      axis=1, keepdims=True)

    pbox = pbox_ref[0]
    gt_d = jnp.concatenate([gl, gt_, gr, gb], axis=1)
    l1 = 0.25 * jnp.abs(pbox - gt_d)
    l1 = jnp.where(gt_d < 0.0, 0.0, l1)
    box_sum = jnp.sum(jnp.sum(l1, axis=0, keepdims=True), axis=1, keepdims=True)

    num = jnp.minimum(gl, gr) * jnp.minimum(gt_, gb)
    den = jnp.maximum(gl, gr) * jnp.maximum(gt_, gb)
    ctr = jnp.sqrt(jnp.clip(num / (den + 1e-8), 0.0, None))
    ctr = jnp.where(gl < 0.0, -1.0, ctr)
    valid = ctr >= 0.0
    tc = jnp.where(valid, ctr, 0.0)
    xl = pctr_ref[0]
    bce = jnp.maximum(xl, 0.0) - xl * tc + jnp.log1p(jnp.exp(-jnp.abs(xl)))
    bce = jnp.where(valid, bce, 0.0)
    ctr_sum = jnp.sum(jnp.sum(bce, axis=0, keepdims=True), axis=1, keepdims=True)

    fg_sum = jnp.sum(jnp.sum(jnp.where(fg, 1.0, 0.0), axis=0, keepdims=True),
                     axis=1, keepdims=True)

    out_ref[0] = jnp.concatenate([cls_sum, box_sum, ctr_sum, fg_sum], axis=1)


def _loss_call(meta, gtt, gt, pcls, pbox, pctr):
    B, L, _ = pcls.shape
    N = gt.shape[1]
    return pl.pallas_call(
        _loss_body,
        out_shape=jax.ShapeDtypeStruct((B, 1, 4), jnp.float32),
        grid=(B,),
        in_specs=[
            pl.BlockSpec((L, 8), lambda b: (0, 0)),
            pl.BlockSpec((1, 5, N), lambda b: (b, 0, 0)),
            pl.BlockSpec((1, N, 5), lambda b: (b, 0, 0)),
            pl.BlockSpec((1, L, _NCLS), lambda b: (b, 0, 0)),
            pl.BlockSpec((1, L, 4), lambda b: (b, 0, 0)),
            pl.BlockSpec((1, L, 1), lambda b: (b, 0, 0)),
        ],
        out_specs=pl.BlockSpec((1, 1, 4), lambda b: (b, 0, 0)),
        compiler_params=pltpu.CompilerParams(
            dimension_semantics=("parallel",),
            vmem_limit_bytes=_VMEM_LIMIT,
        ),
        name="fcos_match_loss",
    )(meta, gtt, gt, pcls, pbox, pctr)


def _build_meta(shapes):
    rows = []
    bounds = [(0.0, _STRIDES[0] * 8.0),
              (_STRIDES[1] * 4.0, _STRIDES[1] * 8.0),
              (_STRIDES[2] * 4.0, float("inf"))]
    for (h, w), s, (lo, hi) in zip(shapes, _STRIDES, bounds):
        ys = (jnp.arange(h, dtype=jnp.float32) + 0.5) * s
        xs = (jnp.arange(w, dtype=jnp.float32) + 0.5) * s
        gy, gx = jnp.meshgrid(ys, xs, indexing="ij")
        L = h * w
        m = jnp.stack([
            gx.ravel(), gy.ravel(),
            jnp.full((L,), float(s), jnp.float32),
            jnp.full((L,), lo, jnp.float32),
            jnp.full((L,), hi, jnp.float32),
            jnp.zeros((L,), jnp.float32),
            jnp.zeros((L,), jnp.float32),
            jnp.zeros((L,), jnp.float32),
        ], axis=-1)
        rows.append(m)
    return jnp.concatenate(rows, axis=0)


def kernel(images, gt_boxes, params):
    B = images.shape[0]

    bbp = params["backbone"]
    w0 = jnp.pad(bbp[0][0].transpose(1, 2, 3, 0).reshape(27, 32),
                 ((0, 5), (0, 0))).astype(jnp.bfloat16)
    c3f = _bb012_call(
        images, w0, bbp[0][1].reshape(1, -1),
        _w_oihw_to_taps(bbp[1][0]).astype(jnp.bfloat16),
        bbp[1][1].reshape(1, -1),
        _w_oihw_to_taps(bbp[2][0]).astype(jnp.bfloat16),
        bbp[2][1].reshape(1, -1))

    feats = [c3f.reshape(B, 64, 64, 64)]
    x = feats[0]
    for i in (3, 4):
        w, b = bbp[i]
        xp = _pad_hw(x, 1)
        xs = _s2d(xp)
        H = xs.shape[1] - 1
        W = xs.shape[2] - 1
        C4 = xs.shape[3]
        Cout = w.shape[0]
        call = _make_chain_call(((2, C4, Cout, True),), H, W, f"bb{i}")
        y = call(xs, [_w_stride2(w).astype(jnp.bfloat16)],
                 [b.reshape(1, -1)])
        x = y.reshape(B, H, W, Cout)
        feats.append(x)

    lats = []
    for f, (w, b) in zip(feats, params["fpn_lat"]):
        H, W, Cin = f.shape[1], f.shape[2], f.shape[3]
        call = _make_chain_call(((1, Cin, 256, False),), H, W, f"lat{H}")
        lats.append(call(f, [_w_oihw_to_taps(w).astype(jnp.bfloat16)],
                         [b.reshape(1, -1)]).reshape(B, H, W, 256))

    p5pre = lats[2]
    p4pre = lats[1] + _up2(p5pre)
    p3pre = lats[0] + _up2(p4pre)

    fpn = []
    for pre, (w, b) in zip([p3pre, p4pre, p5pre], params["fpn_out"]):
        H, W = pre.shape[1], pre.shape[2]
        call = _make_chain_call(((3, 256, 256, False),), H, W, f"fpnout{H}",
                                pad_out=True)
        fpn.append(call(_pad_hw(pre, 1), [_w3(w)],
                        [b.reshape(1, -1)]))

    wstem = jnp.stack([jnp.stack([_w3(w) for (w, _) in params["stem_cls"]]),
                       jnp.stack([_w3(w) for (w, _) in params["stem_box"]])])
    bstem = jnp.stack(
        [jnp.stack([b.reshape(1, -1) for (_, b) in params["stem_cls"]]),
         jnp.stack([b.reshape(1, -1) for (_, b) in params["stem_box"]])])
    wc, bc = params["pred_cls"]
    wb, bbx = params["pred_box"]
    wt, bt = params["pred_ctr"]
    pred_bc_w = jnp.concatenate([_w3(wb), _w3(wt)], axis=-1)
    wpred = jnp.stack([_w3(wc),
                       jnp.pad(pred_bc_w, ((0, 0), (0, 0), (0, 15)))])
    bpred = jnp.stack([bc.reshape(1, -1),
                       jnp.pad(jnp.concatenate([bbx, bt]).reshape(1, -1),
                               ((0, 0), (0, 15)))])

    cls_l, box_l, ctr_l = [], [], []
    for fp in fpn:
        H, W = fp.shape[1] - 2, fp.shape[2] - 2
        head_call = _make_head_call(H, W, f"head{H}")
        out = head_call(fp, wstem, bstem, wpred, bpred)
        cls_l.append(out[:, 0])
        box_l.append(out[:, 1, :, 0:4])
        ctr_l.append(out[:, 1, :, 4:5])

    p_cls = jnp.concatenate(cls_l, axis=1)
    p_box = jnp.concatenate(box_l, axis=1)
    p_ctr = jnp.concatenate(ctr_l, axis=1)

    meta = _build_meta([(f.shape[1] - 2, f.shape[2] - 2) for f in fpn])
    gtt = gt_boxes.transpose(0, 2, 1)
    sums = _loss_call(meta, gtt, gt_boxes, p_cls, p_box, p_ctr)
    tot = jnp.sum(sums[:, 0, :], axis=0)
    norm = jnp.maximum(tot[3], 1.0)
    return tot[0:3] / norm


# speedup vs baseline: 3.8154x; 1.0859x over previous
import jax
import jax.numpy as jnp
from jax import lax
from jax.experimental import pallas as pl
from jax.experimental.pallas import tpu as pltpu

_NCLS = 20
_STRIDES = (8, 16, 32)
_VMEM_LIMIT = 52 * 1024 * 1024


def _make_tiled_conv_call(K, Cin, Cout, relu, H, W, bh, name):
    T = H // bh
    Wp = W + K - 1

    def body(a_ref, b_ref, w_ref, bias_ref, out_ref):
        xin = jnp.concatenate([a_ref[0], b_ref[0][:K - 1]], axis=0)
        acc = jnp.zeros((bh * W, Cout), jnp.float32)
        for kh in range(K):
            for kw in range(K):
                xs = xin[kh:kh + bh, kw:kw + W, :].reshape(bh * W, Cin)
                acc = acc + jnp.dot(xs.astype(jnp.bfloat16), w_ref[kh * K + kw],
                                    preferred_element_type=jnp.float32)
        y = acc + bias_ref[...]
        if relu:
            y = jnp.maximum(y, 0.0)
        out_ref[0] = y

    def call(x, w, bias):
        B = x.shape[0]
        return pl.pallas_call(
            body,
            out_shape=jax.ShapeDtypeStruct((B, H * W, Cout), jnp.float32),
            grid=(B, T),
            in_specs=[
                pl.BlockSpec((1, bh, Wp, Cin), lambda b, t: (b, t, 0, 0)),
                pl.BlockSpec((1, bh, Wp, Cin), lambda b, t: (b, t + 1, 0, 0)),
                pl.BlockSpec((K * K, Cin, Cout), lambda b, t: (0, 0, 0)),
                pl.BlockSpec((1, Cout), lambda b, t: (0, 0)),
            ],
            out_specs=pl.BlockSpec((1, bh * W, Cout), lambda b, t: (b, t, 0)),
            compiler_params=pltpu.CompilerParams(
                dimension_semantics=("parallel", "arbitrary"),
                vmem_limit_bytes=_VMEM_LIMIT,
            ),
            name=name,
        )(x, x, w, bias)

    return call


def _make_chain_call(specs, H, W, name, pad_out=False):
    n = len(specs)
    K0 = specs[0][0]
    C0 = specs[0][1]
    Hp0, Wp0 = H + K0 - 1, W + K0 - 1
    n_scratch = 0 if n == 1 else 2
    Cmid = max(s[2] for s in specs[:-1]) if n > 1 else 0

    def body(x_ref, *refs):
        w_refs = refs[0:2 * n:2]
        b_refs = refs[1:2 * n:2]
        out_ref = refs[2 * n]
        scr = list(refs[2 * n + 1:])
        has_k3 = any(s[0] == 3 for s in specs)
        x3_ref = scr[-1] if has_k3 else None

        for s in (scr[:2] if n > 1 else []):
            s[...] = jnp.zeros_like(s)

        src_ref = x_ref
        src_is_input = True
        for i, (K, Cin, Cout, relu) in enumerate(specs):
            acc = jnp.zeros((H * W, Cout), jnp.float32)
            if K == 3 and x3_ref is not None:
                for kh in range(3):
                    for kw in range(3):
                        if src_is_input:
                            xs = src_ref[0, kh:kh + H, kw:kw + W, :]
                        else:
                            xs = src_ref[kh:kh + H, kw:kw + W, :Cin]
                        x3_ref[:, kw * Cin:(kw + 1) * Cin] = (
                            xs.reshape(H * W, Cin).astype(jnp.bfloat16))
                    acc = acc + jnp.dot(x3_ref[...], w_refs[i][kh],
                                        preferred_element_type=jnp.float32)
            else:
                for kh in range(K):
                    for kw in range(K):
                        if src_is_input:
                            xs = src_ref[0, kh:kh + H, kw:kw + W, :]
                        else:
                            xs = src_ref[kh:kh + H, kw:kw + W, :Cin]
                        xs = xs.reshape(H * W, Cin).astype(jnp.bfloat16)
                        acc = acc + jnp.dot(xs, w_refs[i][kh * K + kw],
                                            preferred_element_type=jnp.float32)
            y = acc + b_refs[i][...]
            if relu:
                y = jnp.maximum(y, 0.0)
            if i == n - 1:
                if pad_out:
                    out_ref[0, 0:1, :, :] = jnp.zeros((1, W + 2, Cout),
                                                      jnp.float32)
                    out_ref[0, H + 1:H + 2, :, :] = jnp.zeros((1, W + 2, Cout),
                                                              jnp.float32)
                    out_ref[0, 1:H + 1, 0:1, :] = jnp.zeros((H, 1, Cout),
                                                            jnp.float32)
                    out_ref[0, 1:H + 1, W + 1:W + 2, :] = jnp.zeros(
                        (H, 1, Cout), jnp.float32)
                    out_ref[0, 1:H + 1, 1:W + 1, :] = y.reshape(H, W, Cout)
                else:
                    out_ref[0] = y
            else:
                dst = scr[i % 2]
                dst[1:H + 1, 1:W + 1, :Cout] = y.reshape(H, W, Cout)
                src_ref = dst
                src_is_input = False

    Cout_last = specs[-1][2]

    def call(x, weights, biases):
        in_specs = [pl.BlockSpec((1, Hp0, Wp0, C0), lambda b: (b, 0, 0, 0))]
        args = [x]
        for (K, Cin, Cout, relu), w, bias in zip(specs, weights, biases):
            in_specs.append(pl.BlockSpec(w.shape, lambda b: (0, 0, 0)))
            in_specs.append(pl.BlockSpec((1, Cout), lambda b: (0, 0)))
            args.append(w)
            args.append(bias)
        scratch = []
        if n_scratch:
            scratch = [pltpu.VMEM((H + 2, W + 2, Cmid), jnp.float32)
                       for _ in range(2)]
        if any(s[0] == 3 for s in specs):
            k3cin = max(s[1] for s in specs if s[0] == 3)
            scratch = scratch + [pltpu.VMEM((H * W, 3 * k3cin), jnp.bfloat16)]
        B = x.shape[0]
        if pad_out:
            out_shape = jax.ShapeDtypeStruct((B, H + 2, W + 2, Cout_last),
                                             jnp.float32)
            out_spec = pl.BlockSpec((1, H + 2, W + 2, Cout_last),
                                    lambda b: (b, 0, 0, 0))
        else:
            out_shape = jax.ShapeDtypeStruct((B, H * W, Cout_last),
                                             jnp.float32)
            out_spec = pl.BlockSpec((1, H * W, Cout_last), lambda b: (b, 0, 0))
        return pl.pallas_call(
            body,
            out_shape=out_shape,
            grid=(B,),
            in_specs=in_specs,
            out_specs=out_spec,
            scratch_shapes=scratch,
            compiler_params=pltpu.CompilerParams(
                dimension_semantics=("parallel",),
                vmem_limit_bytes=_VMEM_LIMIT,
            ),
            name=name,
        )(*args)

    return call


def _make_head_call(H, W, name):
    n_stem = 4

    def body(x_ref, ws_ref, bs_ref, wp_ref, bp_ref, out_ref, s0, s1, x3_ref):
        s0[...] = jnp.zeros_like(s0)
        s1[...] = jnp.zeros_like(s1)
        scr = [s0, s1]
        src_ref = x_ref
        src_is_input = True
        for i in range(n_stem + 1):
            last = i == n_stem
            Cout = 20 if last else 256
            acc = jnp.zeros((H * W, Cout), jnp.float32)
            for kh in range(3):
                for kw in range(3):
                    if src_is_input:
                        xs = src_ref[0, kh:kh + H, kw:kw + W, :]
                    else:
                        xs = src_ref[kh:kh + H, kw:kw + W, :]
                    x3_ref[:, kw * 256:(kw + 1) * 256] = (
                        xs.reshape(H * W, 256).astype(jnp.bfloat16))
                w = wp_ref[0, kh] if last else ws_ref[0, i, kh]
                acc = acc + jnp.dot(x3_ref[...], w,
                                    preferred_element_type=jnp.float32)
            if last:
                out_ref[0, 0] = acc + bp_ref[0]
            else:
                y = jnp.maximum(acc + bs_ref[0, i], 0.0)
                dst = scr[i % 2]
                dst[1:H + 1, 1:W + 1, :] = y.reshape(H, W, 256)
                src_ref = dst
                src_is_input = False

    def call(fp, wstem, bstem, wpred, bpred):
        B = fp.shape[0]
        return pl.pallas_call(
            body,
            out_shape=jax.ShapeDtypeStruct((B, 2, H * W, 20), jnp.float32),
            grid=(B, 2),
            in_specs=[
                pl.BlockSpec((1, H + 2, W + 2, 256), lambda b, r: (b, 0, 0, 0)),
                pl.BlockSpec((1, 4, 3, 768, 256), lambda b, r: (r, 0, 0, 0, 0)),
                pl.BlockSpec((1, 4, 1, 256), lambda b, r: (r, 0, 0, 0)),
                pl.BlockSpec((1, 3, 768, 20), lambda b, r: (r, 0, 0, 0)),
                pl.BlockSpec((1, 1, 20), lambda b, r: (r, 0, 0)),
            ],
            out_specs=pl.BlockSpec((1, 1, H * W, 20), lambda b, r: (b, r, 0, 0)),
            scratch_shapes=[
                pltpu.VMEM((H + 2, W + 2, 256), jnp.float32),
                pltpu.VMEM((H + 2, W + 2, 256), jnp.float32),
                pltpu.VMEM((H * W, 768), jnp.bfloat16),
            ],
            compiler_params=pltpu.CompilerParams(
                dimension_semantics=("parallel", "arbitrary"),
                vmem_limit_bytes=_VMEM_LIMIT,
            ),
            name=name,
        )(fp, wstem, bstem, wpred, bpred)

    return call


_BB_T = 4
_BB_BH3 = 16


def _bb012_body(a_ref, b_ref, w0_ref, b0_ref, w1_ref, b1_ref, w2_ref, b2_ref,
                out_ref, x_sc, p_sc, z_sc, l1_sc, l2_sc):
    t = pl.program_id(1)
    x_sc[:, 0:128, :] = a_ref[0]
    x_sc[:, 128:144, :] = b_ref[0][:, 0:16, :]
    l1_sc[...] = jnp.zeros_like(l1_sc)
    l2_sc[...] = jnp.zeros_like(l2_sc)
    p_sc[27:32, :] = jnp.zeros((5, 512), jnp.float32)

    for q in range(70):
        for ci in range(3):
            for kh in range(3):
                for kw in range(3):
                    p_sc[ci * 9 + kh * 3 + kw, :] = (
                        x_sc[ci, 2 * q + kh + 1, kw:kw + 512])
        z = jax.lax.dot_general(
            p_sc[...].astype(jnp.bfloat16), w0_ref[...],
            (((0,), (0,)), ((), ())),
            preferred_element_type=jnp.float32)
        z_sc[...] = jnp.maximum(z + b0_ref[...], 0.0)
        l1_sc[q, 1:257, 0:32] = z_sc[0:512:2, :]

    @pl.when(t == 0)
    def _():
        l1_sc[0:3, :, :] = jnp.zeros_like(l1_sc[0:3, :, :])

    @pl.when(t == _BB_T - 1)
    def _():
        l1_sc[67:70, :, :] = jnp.zeros_like(l1_sc[67:70, :, :])

    acc1 = jnp.zeros((34 * 128, 64), jnp.float32)
    for kh in range(3):
        for kw in range(3):
            xs = l1_sc[kh:kh + 68:2, kw:kw + 256:2, 0:32]
            xs = xs.reshape(34 * 128, 32).astype(jnp.bfloat16)
            acc1 = acc1 + jnp.dot(xs, w1_ref[kh * 3 + kw],
                                  preferred_element_type=jnp.float32)
    y1 = jnp.maximum(acc1 + b1_ref[...], 0.0).reshape(34, 128, 64)
    l2_sc[0:34, 1:129, 0:64] = y1

    @pl.when(t == 0)
    def _():
        l2_sc[0, :, :] = jnp.zeros_like(l2_sc[0, :, :])

    @pl.when(t == _BB_T - 1)
    def _():
        l2_sc[33, :, :] = jnp.zeros_like(l2_sc[33, :, :])

    acc2 = jnp.zeros((16 * 64, 64), jnp.float32)
    for kh in range(3):
        for kw in range(3):
            xs = l2_sc[kh:kh + 32:2, kw:kw + 128:2, 0:64]
            xs = xs.reshape(16 * 64, 64).astype(jnp.bfloat16)
            acc2 = acc2 + jnp.dot(xs, w2_ref[kh * 3 + kw],
                                  preferred_element_type=jnp.float32)
    out_ref[0] = jnp.maximum(acc2 + b2_ref[...], 0.0)


def _bb012_call(images, w0, b0, w1, b1, w2, b2):
    B = images.shape[0]
    xp = jnp.pad(images, ((0, 0), (0, 0), (8, 120), (1, 1)))
    return pl.pallas_call(
        _bb012_body,
        out_shape=jax.ShapeDtypeStruct((B, 64 * 64, 64), jnp.float32),
        grid=(B, _BB_T),
        in_specs=[
            pl.BlockSpec((1, 3, 128, 514), lambda b, t: (b, 0, t, 0)),
            pl.BlockSpec((1, 3, 128, 514), lambda b, t: (b, 0, t + 1, 0)),
            pl.BlockSpec((32, 32), lambda b, t: (0, 0)),
            pl.BlockSpec((1, 32), lambda b, t: (0, 0)),
            pl.BlockSpec((9, 32, 64), lambda b, t: (0, 0, 0)),
            pl.BlockSpec((1, 64), lambda b, t: (0, 0)),
            pl.BlockSpec((9, 64, 64), lambda b, t: (0, 0, 0)),
            pl.BlockSpec((1, 64), lambda b, t: (0, 0)),
        ],
        out_specs=pl.BlockSpec((1, _BB_BH3 * 64, 64), lambda b, t: (b, t, 0)),
        scratch_shapes=[
            pltpu.VMEM((3, 144, 514), jnp.float32),
            pltpu.VMEM((32, 512), jnp.float32),
            pltpu.VMEM((512, 32), jnp.float32),
            pltpu.VMEM((72, 258, 32), jnp.float32),
            pltpu.VMEM((40, 130, 64), jnp.float32),
        ],
        compiler_params=pltpu.CompilerParams(
            dimension_semantics=("parallel", "arbitrary"),
            vmem_limit_bytes=_VMEM_LIMIT,
        ),
        name="bb012",
    )(xp, xp, w0, b0, w1, b1, w2, b2)


def _to_nhwc(x):
    return x.transpose(0, 2, 3, 1)


def _pad_hw(x, pad):
    return jnp.pad(x, ((0, 0), (pad, pad), (pad, pad), (0, 0)))


def _s2d(x):
    B, H, W, C = x.shape
    x = x.reshape(B, H // 2, 2, W // 2, 2, C)
    x = x.transpose(0, 1, 3, 2, 4, 5)
    return x.reshape(B, H // 2, W // 2, 4 * C)


def _w_oihw_to_taps(w):
    Cout, Cin, K, _ = w.shape
    return w.transpose(2, 3, 1, 0).reshape(K * K, Cin, Cout)


def _w3(w):
    Cout, Cin, _, _ = w.shape
    t = _w_oihw_to_taps(w).reshape(3, 3 * Cin, Cout)
    return t.astype(jnp.bfloat16)


def _w_stride2(w):
    Cout, Cin, _, _ = w.shape
    w = w.transpose(2, 3, 1, 0)
    w = jnp.pad(w, ((0, 1), (0, 1), (0, 0), (0, 0)))
    w = w.reshape(2, 2, 2, 2, Cin, Cout)
    w = w.transpose(0, 2, 1, 3, 4, 5)
    return w.reshape(4, 4 * Cin, Cout)


def _up2(x):
    B, H, W, C = x.shape
    x = jnp.broadcast_to(x[:, :, None, :, None, :], (B, H, 2, W, 2, C))
    return x.reshape(B, 2 * H, 2 * W, C)


def _loss_body(metat_ref, gt_ref, pcls_ref, pbt_ref, out_ref):
    L = metat_ref.shape[1]
    N = gt_ref.shape[1]

    x = metat_ref[0:1, :]
    y = metat_ref[1:2, :]
    stride = metat_ref[2:3, :]
    lower = metat_ref[3:4, :]
    upper = metat_ref[4:5, :]

    gt5 = gt_ref[0]
    x1 = gt5[:, 0:1]
    y1 = gt5[:, 1:2]
    x2 = gt5[:, 2:3]
    y2 = gt5[:, 3:4]

    dl = x - x1
    dt = y - y1
    dr = x2 - x
    db = y2 - y
    mind = jnp.minimum(jnp.minimum(dl, dt), jnp.minimum(dr, db))
    maxd = jnp.maximum(jnp.maximum(dl, dt), jnp.maximum(dr, db))
    inside = mind > 0.0
    fit = (maxd > lower) & (maxd < upper)
    areas = (x2 - x1) * (y2 - y1)
    quality = jnp.where(inside & fit, 1e8 - areas, 0.0)

    q = jnp.max(quality, axis=0, keepdims=True)
    niota = lax.broadcasted_iota(jnp.int32, (N, L), 0)
    sel = jnp.where(quality == q, niota, N)
    idx = jnp.min(sel, axis=0, keepdims=True)
    onehot = niota == idx

    mcols = [jnp.sum(jnp.where(onehot, gt5[:, c:c + 1], 0.0),
                     axis=0, keepdims=True) for c in range(5)]

    bg = q < 1e-5
    mx1 = jnp.where(bg, -1.0, mcols[0])
    my1 = jnp.where(bg, -1.0, mcols[1])
    mx2 = jnp.where(bg, -1.0, mcols[2])
    my2 = jnp.where(bg, -1.0, mcols[3])
    mcls = jnp.where(bg, -1.0, mcols[4])

    gl = jnp.where(bg, -1.0, (x - mx1) / stride)
    gt_ = jnp.where(bg, -1.0, (y - my1) / stride)
    gr = jnp.where(bg, -1.0, (mx2 - x) / stride)
    gb = jnp.where(bg, -1.0, (my2 - y) / stride)

    fg = mcls >= 0.0
    cls_i = jnp.clip(mcls, 0.0, None).astype(jnp.int32)

    logits = pcls_ref[0]
    p = 1.0 / (1.0 + jnp.exp(-logits))
    sp = jnp.log1p(jnp.exp(-jnp.abs(logits)))
    ce0 = jnp.maximum(logits, 0.0) + sp
    ce1 = ce0 - logits
    f0 = 0.75 * ce0 * p * p
    f1 = 0.25 * ce1 * (1.0 - p) * (1.0 - p)
    f0_sum = jnp.sum(jnp.sum(f0, axis=0, keepdims=True),
                     axis=1, keepdims=True)
    ciota = lax.broadcasted_iota(jnp.int32, (_NCLS, L), 0)
    tT = jnp.where((ciota == cls_i) & fg, 1.0, 0.0)
    m = jnp.dot(tT, f1 - f0, preferred_element_type=jnp.float32)
    e1 = lax.broadcasted_iota(jnp.int32, (_NCLS, _NCLS), 0)
    e2 = lax.broadcasted_iota(jnp.int32, (_NCLS, _NCLS), 1)
    tr = jnp.sum(jnp.sum(jnp.where(e1 == e2, m, 0.0), axis=0, keepdims=True),
                 axis=1, keepdims=True)
    cls_sum = f0_sum + tr

    pb = pbt_ref[0]
    gtd4 = jnp.concatenate([gl, gt_, gr, gb], axis=0)
    l1 = 0.25 * jnp.abs(pb[0:4, :] - gtd4)
    l1 = jnp.where(gtd4 < 0.0, 0.0, l1)
    box_sum = jnp.sum(jnp.sum(l1, axis=0, keepdims=True), axis=1, keepdims=True)

    num = jnp.minimum(gl, gr) * jnp.minimum(gt_, gb)
    den = jnp.maximum(gl, gr) * jnp.maximum(gt_, gb)
    ctr = jnp.sqrt(jnp.clip(num / (den + 1e-8), 0.0, None))
    ctr = jnp.where(gl < 0.0, -1.0, ctr)
    valid = ctr >= 0.0
    tc = jnp.where(valid, ctr, 0.0)
    xl = pb[4:5, :]
    bce = jnp.maximum(xl, 0.0) - xl * tc + jnp.log1p(jnp.exp(-jnp.abs(xl)))
    bce = jnp.where(valid, bce, 0.0)
    ctr_sum = jnp.sum(bce, axis=1, keepdims=True)

    fg_sum = jnp.sum(jnp.where(fg, 1.0, 0.0), axis=1, keepdims=True)

    out_ref[0] = jnp.concatenate([cls_sum, box_sum, ctr_sum, fg_sum], axis=1)


def _loss_call(metat, gt, pcls, pbt):
    B, L, _ = pcls.shape
    N = gt.shape[1]
    return pl.pallas_call(
        _loss_body,
        out_shape=jax.ShapeDtypeStruct((B, 1, 4), jnp.float32),
        grid=(B,),
        in_specs=[
            pl.BlockSpec((8, L), lambda b: (0, 0)),
            pl.BlockSpec((1, N, 5), lambda b: (b, 0, 0)),
            pl.BlockSpec((1, L, _NCLS), lambda b: (b, 0, 0)),
            pl.BlockSpec((1, 5, L), lambda b: (b, 0, 0)),
        ],
        out_specs=pl.BlockSpec((1, 1, 4), lambda b: (b, 0, 0)),
        compiler_params=pltpu.CompilerParams(
            dimension_semantics=("parallel",),
            vmem_limit_bytes=_VMEM_LIMIT,
        ),
        name="fcos_match_loss",
    )(metat, gt, pcls, pbt)


def _build_meta(shapes):
    rows = []
    bounds = [(0.0, _STRIDES[0] * 8.0),
              (_STRIDES[1] * 4.0, _STRIDES[1] * 8.0),
              (_STRIDES[2] * 4.0, float("inf"))]
    for (h, w), s, (lo, hi) in zip(shapes, _STRIDES, bounds):
        ys = (jnp.arange(h, dtype=jnp.float32) + 0.5) * s
        xs = (jnp.arange(w, dtype=jnp.float32) + 0.5) * s
        gy, gx = jnp.meshgrid(ys, xs, indexing="ij")
        L = h * w
        m = jnp.stack([
            gx.ravel(), gy.ravel(),
            jnp.full((L,), float(s), jnp.float32),
            jnp.full((L,), lo, jnp.float32),
            jnp.full((L,), hi, jnp.float32),
            jnp.zeros((L,), jnp.float32),
            jnp.zeros((L,), jnp.float32),
            jnp.zeros((L,), jnp.float32),
        ], axis=-1)
        rows.append(m)
    return jnp.concatenate(rows, axis=0)


def kernel(images, gt_boxes, params):
    B = images.shape[0]

    bbp = params["backbone"]
    w0 = jnp.pad(bbp[0][0].transpose(1, 2, 3, 0).reshape(27, 32),
                 ((0, 5), (0, 0))).astype(jnp.bfloat16)
    c3f = _bb012_call(
        images, w0, bbp[0][1].reshape(1, -1),
        _w_oihw_to_taps(bbp[1][0]).astype(jnp.bfloat16),
        bbp[1][1].reshape(1, -1),
        _w_oihw_to_taps(bbp[2][0]).astype(jnp.bfloat16),
        bbp[2][1].reshape(1, -1))

    feats = [c3f.reshape(B, 64, 64, 64)]
    x = feats[0]
    for i in (3, 4):
        w, b = bbp[i]
        xp = _pad_hw(x, 1)
        xs = _s2d(xp)
        H = xs.shape[1] - 1
        W = xs.shape[2] - 1
        C4 = xs.shape[3]
        Cout = w.shape[0]
        call = _make_chain_call(((2, C4, Cout, True),), H, W, f"bb{i}")
        y = call(xs, [_w_stride2(w).astype(jnp.bfloat16)],
                 [b.reshape(1, -1)])
        x = y.reshape(B, H, W, Cout)
        feats.append(x)

    lats = []
    for f, (w, b) in zip(feats, params["fpn_lat"]):
        H, W, Cin = f.shape[1], f.shape[2], f.shape[3]
        call = _make_chain_call(((1, Cin, 256, False),), H, W, f"lat{H}")
        lats.append(call(f, [_w_oihw_to_taps(w).astype(jnp.bfloat16)],
                         [b.reshape(1, -1)]).reshape(B, H, W, 256))

    p5pre = lats[2]
    p4pre = lats[1] + _up2(p5pre)
    p3pre = lats[0] + _up2(p4pre)

    fpn = []
    for pre, (w, b) in zip([p3pre, p4pre, p5pre], params["fpn_out"]):
        H, W = pre.shape[1], pre.shape[2]
        call = _make_chain_call(((3, 256, 256, False),), H, W, f"fpnout{H}",
                                pad_out=True)
        fpn.append(call(_pad_hw(pre, 1), [_w3(w)],
                        [b.reshape(1, -1)]))

    wstem = jnp.stack([jnp.stack([_w3(w) for (w, _) in params["stem_cls"]]),
                       jnp.stack([_w3(w) for (w, _) in params["stem_box"]])])
    bstem = jnp.stack(
        [jnp.stack([b.reshape(1, -1) for (_, b) in params["stem_cls"]]),
         jnp.stack([b.reshape(1, -1) for (_, b) in params["stem_box"]])])
    wc, bc = params["pred_cls"]
    wb, bbx = params["pred_box"]
    wt, bt = params["pred_ctr"]
    pred_bc_w = jnp.concatenate([_w3(wb), _w3(wt)], axis=-1)
    wpred = jnp.stack([_w3(wc),
                       jnp.pad(pred_bc_w, ((0, 0), (0, 0), (0, 15)))])
    bpred = jnp.stack([bc.reshape(1, -1),
                       jnp.pad(jnp.concatenate([bbx, bt]).reshape(1, -1),
                               ((0, 0), (0, 15)))])

    cls_l, bc_l = [], []
    for fp in fpn:
        H, W = fp.shape[1] - 2, fp.shape[2] - 2
        head_call = _make_head_call(H, W, f"head{H}")
        out = head_call(fp, wstem, bstem, wpred, bpred)
        cls_l.append(out[:, 0])
        bc_l.append(out[:, 1])

    p_cls = jnp.concatenate(cls_l, axis=1)
    p_bc = jnp.concatenate(bc_l, axis=1)
    pbt = p_bc.transpose(0, 2, 1)[:, 0:5, :]

    metat = _build_meta([(f.shape[1] - 2, f.shape[2] - 2) for f in fpn]).T
    sums = _loss_call(metat, gt_boxes, p_cls, pbt)
    tot = jnp.sum(sums[:, 0, :], axis=0)
    norm = jnp.maximum(tot[3], 1.0)
    return tot[0:3] / norm
